# baseline (speedup 1.0000x reference)
import functools

import jax
import jax.numpy as jnp
from jax import lax
from jax.experimental import pallas as pl
from jax.experimental.pallas import tpu as pltpu
from jax.experimental.pallas import tpu_sc as plsc

N = 10000
D = 128
E0 = 320000
NP = 10240
NC = 2
NS = 16
NW = NC * NS
PTE = 10368
EP = PTE * NW
CH = 128
NCHUNK = PTE // CH
CHE = 64
NCHE = PTE // CHE
RPT = NP // NS
EBLK = 4096
EGRID = EP // EBLK
NDUMP = NP - N
EPS = 1e-5

_mesh = plsc.VectorSubcoreMesh(core_axis_name="c", subcore_axis_name="s")


def _tc_pre(x_ref, wi, ws, bs, wd, bd, wl, bl, as_o, ad_o, hl_o):
  t = jnp.dot(x_ref[...], wi[...], preferred_element_type=jnp.float32)
  m = jnp.mean(t, axis=0, keepdims=True)
  c = t - m
  v = jnp.mean(c * c, axis=0, keepdims=True)
  h = jnp.maximum(c * lax.rsqrt(v + EPS), 0.0)
  as_o[...] = jnp.dot(h, ws[...], preferred_element_type=jnp.float32) + bs[...]
  ad_o[...] = jnp.dot(h, wd[...], preferred_element_type=jnp.float32) + bd[...]
  hl_o[...] = jnp.dot(h, wl[...], preferred_element_type=jnp.float32) + bl[...]


def _tc_c0(al_ref, out_ref, acc):
  i = pl.program_id(0)

  @pl.when(i == 0)
  def _():
    acc[...] = jnp.zeros_like(acc)

  a = al_ref[...]
  acc[0:1, :] += jnp.sum(a, axis=0, keepdims=True)
  acc[1:2, :] += jnp.sum(a * a, axis=0, keepdims=True)

  @pl.when(i == EGRID - 1)
  def _():
    out_ref[...] = acc[...]


def _tc_c1(al_ref, st, wa1, ba1, a1_o, st_o, acc):
  i = pl.program_id(0)

  @pl.when(i == 0)
  def _():
    acc[...] = jnp.zeros_like(acc)

  a = al_ref[...]
  z = jnp.maximum((a - st[0:1, :]) * st[1:2, :], 0.0)
  a1 = jnp.dot(z, wa1[...], preferred_element_type=jnp.float32) + ba1[...]
  a1_o[...] = a1
  acc[0:1, :] += jnp.sum(a1, axis=0, keepdims=True)
  acc[1:2, :] += jnp.sum(a1 * a1, axis=0, keepdims=True)

  @pl.when(i == EGRID - 1)
  def _():
    st_o[...] = acc[...]


def _tc_c2(a1_ref, st, wa2, ba2, ae_o):
  a1 = a1_ref[...]
  z = jnp.maximum((a1 - st[0:1, :]) * st[1:2, :], 0.0)
  a2 = jnp.dot(z, wa2[...], preferred_element_type=jnp.float32) + ba2[...]
  ae_o[...] = jnp.exp(a2)


def _tc_fin(p0, p1, cnt, x_ref, wo, out_ref):
  s = p0[...] + p1[...]
  o = s / jnp.maximum(cnt[...], 1.0)
  m = jnp.mean(o, axis=0, keepdims=True)
  c = o - m
  v = jnp.mean(c * c, axis=0, keepdims=True)
  o = jnp.maximum(c * lax.rsqrt(v + EPS), 0.0)
  o = jnp.dot(o, wo[...], preferred_element_type=jnp.float32)
  m = jnp.mean(o, axis=0, keepdims=True)
  c = o - m
  v = jnp.mean(c * c, axis=0, keepdims=True)
  o = c * lax.rsqrt(v + EPS)
  out_ref[...] = jnp.maximum(o + x_ref[...], 0.0)


def _wid():
  return lax.axis_index("s") * NC + lax.axis_index("c")


def _pos_rel(px, py, pz, sv, dv):
  rx = plsc.load_gather(px, [dv]) - plsc.load_gather(px, [sv])
  ry = plsc.load_gather(py, [dv]) - plsc.load_gather(py, [sv])
  rz = plsc.load_gather(pz, [dv]) - plsc.load_gather(pz, [sv])
  return rx, ry, rz


def _sc_a(px_h, py_h, pz_h, src_h, dst_h, msk_h, prm_h,
          out_h, t0_h, t1_h, t2_h,
          px, py, pz, sidx, didx, mskv, prm, obuf, tbuf):
  base = _wid() * PTE
  pltpu.sync_copy(px_h, px)
  pltpu.sync_copy(py_h, py)
  pltpu.sync_copy(pz_h, pz)
  pltpu.sync_copy(prm_h, prm)
  va = prm[pl.ds(0, 16)]
  w = [va[i] for i in range(9)]
  bp = [va[9 + i] for i in range(3)]
  zero = jnp.zeros((16,), jnp.float32)

  def chunk(ci, carry):
    off = base + ci * CH
    pltpu.sync_copy(src_h.at[pl.ds(off, CH)], sidx)
    pltpu.sync_copy(dst_h.at[pl.ds(off, CH)], didx)
    pltpu.sync_copy(msk_h.at[pl.ds(off, CH)], mskv)

    def blk(b, c2):
      s0, s1, s2, q0, q1, q2, nl = c2
      sv = sidx[pl.ds(b * 16, 16)]
      dv = didx[pl.ds(b * 16, 16)]
      mv = mskv[pl.ds(b * 16, 16)]
      rx, ry, rz = _pos_rel(px, py, pz, sv, dv)
      t0 = rx * w[0] + ry * w[1] + rz * w[2] + bp[0]
      t1 = rx * w[3] + ry * w[4] + rz * w[5] + bp[1]
      t2 = rx * w[6] + ry * w[7] + rz * w[8] + bp[2]
      tbuf[0, pl.ds(b * 16, 16)] = t0
      tbuf[1, pl.ds(b * 16, 16)] = t1
      tbuf[2, pl.ds(b * 16, 16)] = t2
      tm0 = t0 * mv
      tm1 = t1 * mv
      tm2 = t2 * mv
      return (s0 + tm0, s1 + tm1, s2 + tm2,
              q0 + tm0 * t0, q1 + tm1 * t1, q2 + tm2 * t2, nl + mv)

    carry = lax.fori_loop(0, CH // 16, blk, carry)
    pltpu.sync_copy(tbuf.at[0], t0_h.at[pl.ds(off, CH)])
    pltpu.sync_copy(tbuf.at[1], t1_h.at[pl.ds(off, CH)])
    pltpu.sync_copy(tbuf.at[2], t2_h.at[pl.ds(off, CH)])
    return carry

  acc = lax.fori_loop(0, NCHUNK, chunk, (zero,) * 7)
  for i in range(7):
    obuf[pl.ds(i * 16, 16)] = acc[i]
  pltpu.sync_copy(obuf, out_h.at[_wid()])


def _sc_b(asrc_h, adst_h, src_h, dst_h, msk_h, t0_h, t1_h, t2_h, prm_h, wp2_h,
          alpha_h,
          sidx, didx, mskv, tb, prm, wp2, bufS, bufD, abuf,
          semS0, semS1, semD0, semD1):
  base = _wid() * PTE
  pltpu.sync_copy(prm_h, prm)
  pltpu.sync_copy(wp2_h, wp2)
  vb = prm[pl.ds(16, 16)]
  m1 = [vb[i] for i in range(3)]
  iv1 = [vb[3 + i] for i in range(3)]
  w2 = [[wp2[j, pl.ds(p * 16, 16)] for p in range(8)] for j in range(4)]
  th = [t0_h, t1_h, t2_h]
  semS = [semS0, semS1]
  semD = [semD0, semD1]

  def issue(g, k):
    off = base + g * CH
    pltpu.sync_copy(src_h.at[pl.ds(off, CH)], sidx.at[k])
    pltpu.sync_copy(dst_h.at[pl.ds(off, CH)], didx.at[k])
    pltpu.sync_copy(msk_h.at[pl.ds(off, CH)], mskv.at[k])
    for j in range(3):
      pltpu.sync_copy(th[j].at[pl.ds(off, CH)], tb.at[k, j])
    pltpu.async_copy(asrc_h.at[sidx.at[k]], bufS.at[k], semS[k])
    pltpu.async_copy(adst_h.at[didx.at[k]], bufD.at[k], semD[k])

  def waitg(k):
    pltpu.make_async_copy(asrc_h.at[sidx.at[k]], bufS.at[k], semS[k]).wait()
    pltpu.make_async_copy(adst_h.at[didx.at[k]], bufD.at[k], semD[k]).wait()

  def compute(g, k):
    off = base + g * CH

    def blk(b, __):
      mv = mskv[k, pl.ds(b * 16, 16)]
      rr = []
      for j in range(3):
        t = tb[k, j, pl.ds(b * 16, 16)]
        rr.append(jnp.maximum((t - m1[j]) * iv1[j], 0.0))
      for e in range(16):
        row = b * 16 + e
        r0 = rr[0][e]
        r1 = rr[1][e]
        r2 = rr[2][e]
        me = mv[e]
        for p in range(8):
          dp = r0 * w2[0][p] + r1 * w2[1][p] + r2 * w2[2][p] + w2[3][p]
          al = (bufD[k, row, pl.ds(p * 16, 16)]
                - bufS[k, row, pl.ds(p * 16, 16)] + dp) * me
          abuf[row, pl.ds(p * 16, 16)] = al
      return 0

    lax.fori_loop(0, CH // 16, blk, 0)
    pltpu.sync_copy(abuf, alpha_h.at[pl.ds(off, CH)])

  issue(0, 0)

  def pair(ci, _):
    g0 = 2 * ci
    issue(g0 + 1, 1)
    waitg(0)
    compute(g0, 0)
    issue(g0 + 2, 0)
    waitg(1)
    compute(g0 + 1, 1)
    return 0

  lax.fori_loop(0, NCHUNK // 2, pair, 0)
  waitg(0)
  compute(NCHUNK - 1, 0)


def _sc_d(ae_h, dst_h, msk_h, asum_o, cnt_o,
          asum_sh, cnt_sh, aev, didx, mskv, zb, zb1):
  cid = lax.axis_index("c")
  sid = lax.axis_index("s")
  base = _wid() * PTE
  zv = jnp.zeros((16,), jnp.float32)

  def zrow(i, _):
    zb[i, :] = zv
    return 0

  lax.fori_loop(0, RPT, zrow, 0)

  def zrow1(i, _):
    zb1[pl.ds(i * 16, 16)] = zv
    return 0

  lax.fori_loop(0, RPT // 16, zrow1, 0)
  pltpu.sync_copy(zb, asum_sh.at[pl.ds(sid * RPT, RPT)])
  pltpu.sync_copy(zb1, cnt_sh.at[pl.ds(sid * RPT, RPT)])
  plsc.subcore_barrier()

  def chunk(ci, _):
    off = base + ci * CH
    pltpu.sync_copy(ae_h.at[pl.ds(off, CH)], aev)
    pltpu.sync_copy(dst_h.at[pl.ds(off, CH)], didx)
    pltpu.sync_copy(msk_h.at[pl.ds(off, CH)], mskv)
    pltpu.sync_copy(aev, asum_sh.at[didx], add=True)
    pltpu.sync_copy(mskv, cnt_sh.at[didx], add=True)
    return 0

  lax.fori_loop(0, NCHUNK, chunk, 0)
  plsc.subcore_barrier()
  pltpu.sync_copy(asum_sh.at[pl.ds(sid * RPT, RPT)], zb)
  pltpu.sync_copy(zb, asum_o.at[cid, pl.ds(sid * RPT, RPT)])
  pltpu.sync_copy(cnt_sh.at[pl.ds(sid * RPT, RPT)], zb1)
  pltpu.sync_copy(zb1, cnt_o.at[cid, pl.ds(sid * RPT, RPT)])


def _sc_e(hlin_h, ae_h, asum_h, t0_h, t1_h, t2_h, src_h, dst_h, prm_h, wp2_h,
          out_o,
          out_sh, sidx, didx, tb, prm, wp2, aev, asv, bufH, msgb,
          semH0, semH1, semA0, semA1):
  cid = lax.axis_index("c")
  sid = lax.axis_index("s")
  base = _wid() * PTE
  pltpu.sync_copy(prm_h, prm)
  pltpu.sync_copy(wp2_h, wp2)
  vb = prm[pl.ds(16, 16)]
  m1 = [vb[i] for i in range(3)]
  iv1 = [vb[3 + i] for i in range(3)]
  w2 = [[wp2[j, pl.ds(p * 16, 16)] for p in range(8)] for j in range(4)]
  th = [t0_h, t1_h, t2_h]
  semH = [semH0, semH1]
  semA = [semA0, semA1]
  zv = jnp.zeros((16,), jnp.float32)

  def zrow(i, _):
    for p in range(8):
      msgb[i, pl.ds(p * 16, 16)] = zv
    return 0

  lax.fori_loop(0, CHE, zrow, 0)
  for r in range(RPT // CHE):
    pltpu.sync_copy(msgb, out_sh.at[pl.ds(sid * RPT + r * CHE, CHE)])
  plsc.subcore_barrier()

  def issue(g, k):
    off = base + g * CHE
    pltpu.sync_copy(src_h.at[pl.ds(off, CHE)], sidx.at[k])
    pltpu.sync_copy(dst_h.at[pl.ds(off, CHE)], didx.at[k])
    pltpu.sync_copy(ae_h.at[pl.ds(off, CHE)], aev.at[k])
    for j in range(3):
      pltpu.sync_copy(th[j].at[pl.ds(off, CHE)], tb.at[k, j])
    pltpu.async_copy(hlin_h.at[sidx.at[k]], bufH.at[k], semH[k])
    pltpu.async_copy(asum_h.at[didx.at[k]], asv.at[k], semA[k])

  def waitg(k):
    pltpu.make_async_copy(hlin_h.at[sidx.at[k]], bufH.at[k], semH[k]).wait()
    pltpu.make_async_copy(asum_h.at[didx.at[k]], asv.at[k], semA[k]).wait()

  def compute(g, k):
    def blk(b, __):
      rr = []
      for j in range(3):
        t = tb[k, j, pl.ds(b * 16, 16)]
        rr.append(jnp.maximum((t - m1[j]) * iv1[j], 0.0))
      for e in range(16):
        row = b * 16 + e
        r0 = rr[0][e]
        r1 = rr[1][e]
        r2 = rr[2][e]
        att = aev[k, row, :] / (asv[k, row, :] + 1e-16)
        for p in range(8):
          dp = r0 * w2[0][p] + r1 * w2[1][p] + r2 * w2[2][p] + w2[3][p]
          msgb[row, pl.ds(p * 16, 16)] = att * (
              bufH[k, row, pl.ds(p * 16, 16)] + dp)
      return 0

    lax.fori_loop(0, CHE // 16, blk, 0)
    pltpu.sync_copy(msgb, out_sh.at[didx.at[k]], add=True)

  issue(0, 0)

  def pair(ci, _):
    g0 = 2 * ci
    issue(g0 + 1, 1)
    waitg(0)
    compute(g0, 0)
    issue(g0 + 2, 0)
    waitg(1)
    compute(g0 + 1, 1)
    return 0

  lax.fori_loop(0, NCHE // 2 - 1, pair, 0)
  issue(NCHE - 1, 1)
  waitg(0)
  compute(NCHE - 2, 0)
  waitg(1)
  compute(NCHE - 1, 1)
  plsc.subcore_barrier()
  for r in range(RPT // CHE):
    pltpu.sync_copy(out_sh.at[pl.ds(sid * RPT + r * CHE, CHE)], msgb)
    pltpu.sync_copy(msgb, out_o.at[cid, pl.ds(sid * RPT + r * CHE, CHE)])


def kernel(x, pos, edge_index, W_in, W_out, W_lin, b_lin, W_src, b_src,
           W_dst, b_dst, Wp1, bp1, Wp2, bp2, Wa1, ba1, Wa2, ba2):
  f32 = jnp.float32

  src0, dst0 = edge_index[0], edge_index[1]
  keep = src0 != dst0
  loops = jnp.arange(N, dtype=jnp.int32)
  npad = EP - (E0 + N)
  dump0 = N + (jnp.arange(E0, dtype=jnp.int32) % NDUMP)
  dumpP = N + (jnp.arange(npad, dtype=jnp.int32) % NDUMP)
  src = jnp.concatenate([src0, loops, jnp.zeros((npad,), jnp.int32)])
  dst = jnp.concatenate([jnp.where(keep, dst0, dump0), loops, dumpP])
  msk = jnp.concatenate([keep.astype(f32), jnp.ones((N,), f32),
                         jnp.zeros((npad,), f32)])

  sds = jax.ShapeDtypeStruct
  a_src, a_dst, h_lin = pl.pallas_call(
      _tc_pre,
      out_shape=[sds((N, D), f32)] * 3,
  )(x, W_in.T, W_src.T, b_src[None, :], W_dst.T, b_dst[None, :],
    W_lin.T, b_lin[None, :])

  pad_n = lambda a: jnp.pad(a, ((0, NP - N), (0, 0)))
  a_src_p = pad_n(a_src)
  a_dst_p = pad_n(a_dst)
  hlin_p = pad_n(h_lin)
  posx = jnp.pad(pos[:, 0], (0, NP - N))
  posy = jnp.pad(pos[:, 1], (0, NP - N))
  posz = jnp.pad(pos[:, 2], (0, NP - N))

  prmA = jnp.concatenate([Wp1.reshape(-1), bp1, jnp.zeros((20,), f32)])
  partA, t0a, t1a, t2a = pl.kernel(
      _sc_a,
      out_type=[sds((NW, 112), f32), sds((EP,), f32), sds((EP,), f32),
                sds((EP,), f32)],
      mesh=_mesh,
      compiler_params=pltpu.CompilerParams(needs_layout_passes=False, use_tc_tiling_on_sc=False),
      scratch_types=[
          pltpu.VMEM((NP,), f32), pltpu.VMEM((NP,), f32),
          pltpu.VMEM((NP,), f32),
          pltpu.VMEM((CH,), jnp.int32), pltpu.VMEM((CH,), jnp.int32),
          pltpu.VMEM((CH,), f32),
          pltpu.VMEM((32,), f32), pltpu.VMEM((112,), f32),
          pltpu.VMEM((3, CH), f32),
      ],
  )(posx, posy, posz, src, dst, msk, prmA)

  tot = jnp.sum(partA.reshape(NW, 7, 16), axis=(0, 2))
  n_live = tot[6]
  m1 = tot[0:3] / n_live
  v1 = tot[3:6] / n_live - m1 * m1
  iv1 = lax.rsqrt(v1 + EPS)

  prmB = jnp.concatenate([Wp1.reshape(-1), bp1, jnp.zeros((4,), f32),
                          m1, iv1, jnp.zeros((10,), f32)])
  wp2t = jnp.concatenate([Wp2.T, bp2[None, :]])
  alpha = pl.kernel(
      _sc_b,
      out_type=sds((EP, D), f32),
      mesh=_mesh,
      compiler_params=pltpu.CompilerParams(needs_layout_passes=False, use_tc_tiling_on_sc=False),
      scratch_types=[
          pltpu.VMEM((2, CH), jnp.int32), pltpu.VMEM((2, CH), jnp.int32),
          pltpu.VMEM((2, CH), f32), pltpu.VMEM((2, 3, CH), f32),
          pltpu.VMEM((32,), f32), pltpu.VMEM((4, D), f32),
          pltpu.VMEM((2, CH, D), f32), pltpu.VMEM((2, CH, D), f32),
          pltpu.VMEM((CH, D), f32),
          pltpu.SemaphoreType.DMA, pltpu.SemaphoreType.DMA,
          pltpu.SemaphoreType.DMA, pltpu.SemaphoreType.DMA,
      ],
  )(a_src_p, a_dst_p, src, dst, msk, t0a, t1a, t2a, prmB, wp2t)

  stats2 = pl.pallas_call(
      _tc_c0,
      grid=(EGRID,),
      in_specs=[pl.BlockSpec((EBLK, D), lambda i: (i, 0))],
      out_specs=pl.BlockSpec((8, D), lambda i: (0, 0)),
      out_shape=sds((8, D), f32),
      scratch_shapes=[pltpu.VMEM((8, D), f32)],
  )(alpha)

  m2 = stats2[0] / n_live
  v2 = stats2[1] / n_live - m2 * m2
  iv2 = lax.rsqrt(v2 + EPS)
  st2 = jnp.zeros((8, D), f32).at[0].set(m2).at[1].set(iv2)

  DA = D // 8
  a1, stats3r = pl.pallas_call(
      _tc_c1,
      grid=(EGRID,),
      in_specs=[
          pl.BlockSpec((EBLK, D), lambda i: (i, 0)),
          pl.BlockSpec((8, D), lambda i: (0, 0)),
          pl.BlockSpec((D, DA), lambda i: (0, 0)),
          pl.BlockSpec((1, DA), lambda i: (0, 0)),
      ],
      out_specs=[
          pl.BlockSpec((EBLK, DA), lambda i: (i, 0)),
          pl.BlockSpec((8, DA), lambda i: (0, 0)),
      ],
      out_shape=[sds((EP, DA), f32), sds((8, DA), f32)],
      scratch_shapes=[pltpu.VMEM((8, DA), f32)],
  )(alpha, st2, Wa1.T, ba1[None, :])

  cdead = jnp.maximum((0.0 - m2) * iv2, 0.0) @ Wa1.T + ba1
  n_dead = jnp.float32(EP) - n_live
  s3 = stats3r[0] - n_dead * cdead
  q3 = stats3r[1] - n_dead * cdead * cdead
  m3 = s3 / n_live
  v3 = q3 / n_live - m3 * m3
  iv3 = lax.rsqrt(v3 + EPS)
  st3 = jnp.zeros((8, DA), f32).at[0].set(m3).at[1].set(iv3)

  ae = pl.pallas_call(
      _tc_c2,
      grid=(EGRID,),
      in_specs=[
          pl.BlockSpec((EBLK, DA), lambda i: (i, 0)),
          pl.BlockSpec((8, DA), lambda i: (0, 0)),
          pl.BlockSpec((DA, DA), lambda i: (0, 0)),
          pl.BlockSpec((1, DA), lambda i: (0, 0)),
      ],
      out_specs=pl.BlockSpec((EBLK, DA), lambda i: (i, 0)),
      out_shape=sds((EP, DA), f32),
  )(a1, st3, Wa2.T, ba2[None, :])

  asum_p, cnt_p = pl.kernel(
      _sc_d,
      out_type=[sds((NC, NP, DA), f32), sds((NC, NP), f32)],
      mesh=_mesh,
      compiler_params=pltpu.CompilerParams(needs_layout_passes=False, use_tc_tiling_on_sc=False),
      scratch_types=[
          pltpu.VMEM_SHARED((NP, DA), f32), pltpu.VMEM_SHARED((NP,), f32),
          pltpu.VMEM((CH, DA), f32),
          pltpu.VMEM((CH,), jnp.int32), pltpu.VMEM((CH,), f32),
          pltpu.VMEM((RPT, DA), f32), pltpu.VMEM((RPT,), f32),
      ],
  )(ae, dst, msk)

  asum = asum_p[0] + asum_p[1]
  cnt = cnt_p[0] + cnt_p[1]

  out_p = pl.kernel(
      _sc_e,
      out_type=sds((NC, NP, D), f32),
      mesh=_mesh,
      compiler_params=pltpu.CompilerParams(needs_layout_passes=False, use_tc_tiling_on_sc=False),
      scratch_types=[
          pltpu.VMEM_SHARED((NP, D), f32),
          pltpu.VMEM((2, CHE), jnp.int32), pltpu.VMEM((2, CHE), jnp.int32),
          pltpu.VMEM((2, 3, CHE), f32),
          pltpu.VMEM((32,), f32), pltpu.VMEM((4, D), f32),
          pltpu.VMEM((2, CHE, DA), f32), pltpu.VMEM((2, CHE, DA), f32),
          pltpu.VMEM((2, CHE, D), f32), pltpu.VMEM((CHE, D), f32),
          pltpu.SemaphoreType.DMA, pltpu.SemaphoreType.DMA,
          pltpu.SemaphoreType.DMA, pltpu.SemaphoreType.DMA,
      ],
  )(hlin_p, ae, asum, t0a, t1a, t2a, src, dst, prmB, wp2t)

  out = pl.pallas_call(
      _tc_fin,
      out_shape=sds((N, D), f32),
  )(out_p[0, :N], out_p[1, :N], cnt[:N, None], x, W_out.T)
  return out


# speedup vs baseline: 1.3202x; 1.3202x over previous
import functools

import jax
import jax.numpy as jnp
from jax import lax
from jax.experimental import pallas as pl
from jax.experimental.pallas import tpu as pltpu
from jax.experimental.pallas import tpu_sc as plsc

N = 10000
D = 128
E0 = 320000
NP = 10240
NC = 2
NS = 16
NW = NC * NS
PTE = 10368
EP = PTE * NW
CH = 128
NCHUNK = PTE // CH
CHB = 96
NCHB = PTE // CHB
CHE = 96
NCHE = PTE // CHE
RPT = NP // NS
EBLK = 4096
EGRID = EP // EBLK
NDUMP = NP - N
EPS = 1e-5

_mesh = plsc.VectorSubcoreMesh(core_axis_name="c", subcore_axis_name="s")


def _tc_pre(x_ref, wi, ws, bs, wd, bd, wl, bl, as_o, ad_o, hl_o):
  t = jnp.dot(x_ref[...], wi[...], preferred_element_type=jnp.float32)
  m = jnp.mean(t, axis=0, keepdims=True)
  c = t - m
  v = jnp.mean(c * c, axis=0, keepdims=True)
  h = jnp.maximum(c * lax.rsqrt(v + EPS), 0.0)
  as_o[...] = jnp.dot(h, ws[...], preferred_element_type=jnp.float32) + bs[...]
  ad_o[...] = jnp.dot(h, wd[...], preferred_element_type=jnp.float32) + bd[...]
  hl_o[...] = jnp.dot(h, wl[...], preferred_element_type=jnp.float32) + bl[...]


def _tc_c0(al_ref, out_ref, acc):
  i = pl.program_id(0)

  @pl.when(i == 0)
  def _():
    acc[...] = jnp.zeros_like(acc)

  a = al_ref[...]
  acc[0:1, :] += jnp.sum(a, axis=0, keepdims=True)
  acc[1:2, :] += jnp.sum(a * a, axis=0, keepdims=True)

  @pl.when(i == EGRID - 1)
  def _():
    out_ref[...] = acc[...]


def _tc_c1(al_ref, st, wa1, ba1, a1_o, st_o, acc):
  i = pl.program_id(0)

  @pl.when(i == 0)
  def _():
    acc[...] = jnp.zeros_like(acc)

  a = al_ref[...]
  z = jnp.maximum((a - st[0:1, :]) * st[1:2, :], 0.0)
  a1 = jnp.dot(z, wa1[...], preferred_element_type=jnp.float32) + ba1[...]
  a1_o[...] = a1
  acc[0:1, :] += jnp.sum(a1, axis=0, keepdims=True)
  acc[1:2, :] += jnp.sum(a1 * a1, axis=0, keepdims=True)

  @pl.when(i == EGRID - 1)
  def _():
    st_o[...] = acc[...]


def _tc_c2(a1_ref, st, wa2, ba2, ae_o):
  a1 = a1_ref[...]
  z = jnp.maximum((a1 - st[0:1, :]) * st[1:2, :], 0.0)
  a2 = jnp.dot(z, wa2[...], preferred_element_type=jnp.float32) + ba2[...]
  ae_o[...] = jnp.exp(a2)


def _tc_fin(p0, p1, cnt, x_ref, wo, out_ref):
  s = p0[...] + p1[...]
  o = s / jnp.maximum(cnt[...], 1.0)
  m = jnp.mean(o, axis=0, keepdims=True)
  c = o - m
  v = jnp.mean(c * c, axis=0, keepdims=True)
  o = jnp.maximum(c * lax.rsqrt(v + EPS), 0.0)
  o = jnp.dot(o, wo[...], preferred_element_type=jnp.float32)
  m = jnp.mean(o, axis=0, keepdims=True)
  c = o - m
  v = jnp.mean(c * c, axis=0, keepdims=True)
  o = c * lax.rsqrt(v + EPS)
  out_ref[...] = jnp.maximum(o + x_ref[...], 0.0)


def _wid():
  return lax.axis_index("s") * NC + lax.axis_index("c")


def _pos_rel(px, py, pz, sv, dv):
  rx = plsc.load_gather(px, [dv]) - plsc.load_gather(px, [sv])
  ry = plsc.load_gather(py, [dv]) - plsc.load_gather(py, [sv])
  rz = plsc.load_gather(pz, [dv]) - plsc.load_gather(pz, [sv])
  return rx, ry, rz


def _sc_a(px_h, py_h, pz_h, src_h, dst_h, msk_h, prm_h,
          out_h, t0_h, t1_h, t2_h,
          px, py, pz, sidx, didx, mskv, prm, obuf, tbuf):
  base = _wid() * PTE
  pltpu.sync_copy(px_h, px)
  pltpu.sync_copy(py_h, py)
  pltpu.sync_copy(pz_h, pz)
  pltpu.sync_copy(prm_h, prm)
  va = prm[pl.ds(0, 16)]
  w = [va[i] for i in range(9)]
  bp = [va[9 + i] for i in range(3)]
  zero = jnp.zeros((16,), jnp.float32)

  def chunk(ci, carry):
    off = base + ci * CH
    pltpu.sync_copy(src_h.at[pl.ds(off, CH)], sidx)
    pltpu.sync_copy(dst_h.at[pl.ds(off, CH)], didx)
    pltpu.sync_copy(msk_h.at[pl.ds(off, CH)], mskv)

    def blk(b, c2):
      s0, s1, s2, q0, q1, q2, nl = c2
      sv = sidx[pl.ds(b * 16, 16)]
      dv = didx[pl.ds(b * 16, 16)]
      mv = mskv[pl.ds(b * 16, 16)]
      rx, ry, rz = _pos_rel(px, py, pz, sv, dv)
      t0 = rx * w[0] + ry * w[1] + rz * w[2] + bp[0]
      t1 = rx * w[3] + ry * w[4] + rz * w[5] + bp[1]
      t2 = rx * w[6] + ry * w[7] + rz * w[8] + bp[2]
      tbuf[0, pl.ds(b * 16, 16)] = t0
      tbuf[1, pl.ds(b * 16, 16)] = t1
      tbuf[2, pl.ds(b * 16, 16)] = t2
      tm0 = t0 * mv
      tm1 = t1 * mv
      tm2 = t2 * mv
      return (s0 + tm0, s1 + tm1, s2 + tm2,
              q0 + tm0 * t0, q1 + tm1 * t1, q2 + tm2 * t2, nl + mv)

    carry = lax.fori_loop(0, CH // 16, blk, carry)
    pltpu.sync_copy(tbuf.at[0], t0_h.at[pl.ds(off, CH)])
    pltpu.sync_copy(tbuf.at[1], t1_h.at[pl.ds(off, CH)])
    pltpu.sync_copy(tbuf.at[2], t2_h.at[pl.ds(off, CH)])
    return carry

  acc = lax.fori_loop(0, NCHUNK, chunk, (zero,) * 7)
  for i in range(7):
    obuf[pl.ds(i * 16, 16)] = acc[i]
  pltpu.sync_copy(obuf, out_h.at[_wid()])


def _sc_b(asrc_h, adst_h, src_h, dst_h, msk_h, t0_h, t1_h, t2_h, prm_h, wp2_h,
          alpha_h,
          sall, dall, mall, tall, prm, wp2, bufS, bufD, abuf,
          semS0, semS1, semD0, semD1):
  base = _wid() * PTE
  pltpu.sync_copy(prm_h, prm)
  pltpu.sync_copy(wp2_h, wp2)
  pltpu.sync_copy(src_h.at[pl.ds(base, PTE)], sall)
  pltpu.sync_copy(dst_h.at[pl.ds(base, PTE)], dall)
  pltpu.sync_copy(msk_h.at[pl.ds(base, PTE)], mall)
  pltpu.sync_copy(t0_h.at[pl.ds(base, PTE)], tall.at[0])
  pltpu.sync_copy(t1_h.at[pl.ds(base, PTE)], tall.at[1])
  pltpu.sync_copy(t2_h.at[pl.ds(base, PTE)], tall.at[2])
  vb = prm[pl.ds(16, 16)]
  m1 = [vb[i] for i in range(3)]
  iv1 = [vb[3 + i] for i in range(3)]
  w2 = [[wp2[j, pl.ds(p * 16, 16)] for p in range(8)] for j in range(4)]
  semS = [semS0, semS1]
  semD = [semD0, semD1]

  def issue(g, k):
    pltpu.async_copy(asrc_h.at[sall.at[pl.ds(g * CHB, CHB)]],
                     bufS.at[k], semS[k])
    pltpu.async_copy(adst_h.at[dall.at[pl.ds(g * CHB, CHB)]],
                     bufD.at[k], semD[k])

  def waitg(g, k):
    pltpu.make_async_copy(asrc_h.at[sall.at[pl.ds(g * CHB, CHB)]],
                          bufS.at[k], semS[k]).wait()
    pltpu.make_async_copy(adst_h.at[dall.at[pl.ds(g * CHB, CHB)]],
                          bufD.at[k], semD[k]).wait()

  def compute(g, k):
    def blk(b, __):
      lo = g * CHB + b * 16
      mv = mall[pl.ds(lo, 16)]
      rr = []
      for j in range(3):
        t = tall[j, pl.ds(lo, 16)]
        rr.append(jnp.maximum((t - m1[j]) * iv1[j], 0.0))
      for e in range(16):
        row = b * 16 + e
        r0 = rr[0][e]
        r1 = rr[1][e]
        r2 = rr[2][e]
        me = mv[e]
        for p in range(8):
          dp = r0 * w2[0][p] + r1 * w2[1][p] + r2 * w2[2][p] + w2[3][p]
          al = (bufD[k, row, pl.ds(p * 16, 16)]
                - bufS[k, row, pl.ds(p * 16, 16)] + dp) * me
          abuf[row, pl.ds(p * 16, 16)] = al
      return 0

    lax.fori_loop(0, CHB // 16, blk, 0)
    pltpu.sync_copy(abuf, alpha_h.at[pl.ds(base + g * CHB, CHB)])

  issue(0, 0)

  def pair(ci, _):
    g0 = 2 * ci
    issue(g0 + 1, 1)
    waitg(g0, 0)
    compute(g0, 0)
    issue(g0 + 2, 0)
    waitg(g0 + 1, 1)
    compute(g0 + 1, 1)
    return 0

  lax.fori_loop(0, NCHB // 2 - 1, pair, 0)
  issue(NCHB - 1, 1)
  waitg(NCHB - 2, 0)
  compute(NCHB - 2, 0)
  waitg(NCHB - 1, 1)
  compute(NCHB - 1, 1)


def _sc_d(ae_h, dst_h, msk_h, asum_o, cnt_o,
          asum_sh, cnt_sh, aev, didx, mskv, zb, zb1):
  cid = lax.axis_index("c")
  sid = lax.axis_index("s")
  base = _wid() * PTE
  zv = jnp.zeros((16,), jnp.float32)

  def zrow(i, _):
    zb[i, :] = zv
    return 0

  lax.fori_loop(0, RPT, zrow, 0)

  def zrow1(i, _):
    zb1[pl.ds(i * 16, 16)] = zv
    return 0

  lax.fori_loop(0, RPT // 16, zrow1, 0)
  pltpu.sync_copy(zb, asum_sh.at[pl.ds(sid * RPT, RPT)])
  pltpu.sync_copy(zb1, cnt_sh.at[pl.ds(sid * RPT, RPT)])
  plsc.subcore_barrier()

  def chunk(ci, _):
    off = base + ci * CH
    pltpu.sync_copy(ae_h.at[pl.ds(off, CH)], aev)
    pltpu.sync_copy(dst_h.at[pl.ds(off, CH)], didx)
    pltpu.sync_copy(msk_h.at[pl.ds(off, CH)], mskv)
    pltpu.sync_copy(aev, asum_sh.at[didx], add=True)
    pltpu.sync_copy(mskv, cnt_sh.at[didx], add=True)
    return 0

  lax.fori_loop(0, NCHUNK, chunk, 0)
  plsc.subcore_barrier()
  pltpu.sync_copy(asum_sh.at[pl.ds(sid * RPT, RPT)], zb)
  pltpu.sync_copy(zb, asum_o.at[cid, pl.ds(sid * RPT, RPT)])
  pltpu.sync_copy(cnt_sh.at[pl.ds(sid * RPT, RPT)], zb1)
  pltpu.sync_copy(zb1, cnt_o.at[cid, pl.ds(sid * RPT, RPT)])


def _sc_e(hlin_h, ae_h, asum_h, t0_h, t1_h, t2_h, src_h, dst_h, prm_h, wp2_h,
          out_o,
          out_sh, sstg, dstg, tstg, aestg, prm, wp2, asv, bufH, msgb,
          semH0, semH1, semA0, semA1, semT0, semT1, semT2):
  cid = lax.axis_index("c")
  sid = lax.axis_index("s")
  base = _wid() * PTE
  pltpu.sync_copy(prm_h, prm)
  pltpu.sync_copy(wp2_h, wp2)
  vb = prm[pl.ds(16, 16)]
  m1 = [vb[i] for i in range(3)]
  iv1 = [vb[3 + i] for i in range(3)]
  w2 = [[wp2[j, pl.ds(p * 16, 16)] for p in range(8)] for j in range(4)]
  th = [t0_h, t1_h, t2_h]
  semH = [semH0, semH1]
  semA = [semA0, semA1]
  semT = [semT0, semT1, semT2]
  zv = jnp.zeros((16,), jnp.float32)

  def zrow(i, _):
    for p in range(8):
      msgb[i, pl.ds(p * 16, 16)] = zv
    return 0

  lax.fori_loop(0, CHE, zrow, 0)
  for r in range(RPT // 80):
    pltpu.sync_copy(msgb.at[pl.ds(0, 80)],
                    out_sh.at[pl.ds(sid * RPT + r * 80, 80)])
  plsc.subcore_barrier()

  def stg_copies(g, s):
    off = base + g * CHE
    return [
        (src_h.at[pl.ds(off, CHE)], sstg.at[s]),
        (dst_h.at[pl.ds(off, CHE)], dstg.at[s]),
        (ae_h.at[pl.ds(off, CHE)], aestg.at[s]),
        (th[0].at[pl.ds(off, CHE)], tstg.at[s, 0]),
        (th[1].at[pl.ds(off, CHE)], tstg.at[s, 1]),
        (th[2].at[pl.ds(off, CHE)], tstg.at[s, 2]),
    ]

  def stage(g, s):
    for a, b in stg_copies(g, s):
      pltpu.async_copy(a, b, semT[s])

  def waitstage(g, s):
    for a, b in stg_copies(g, s):
      pltpu.make_async_copy(a, b, semT[s]).wait()

  def gather(s, k):
    pltpu.async_copy(hlin_h.at[sstg.at[s]], bufH.at[k], semH[k])
    pltpu.async_copy(asum_h.at[dstg.at[s]], asv.at[k], semA[k])

  def waitgather(s, k):
    pltpu.make_async_copy(hlin_h.at[sstg.at[s]], bufH.at[k], semH[k]).wait()
    pltpu.make_async_copy(asum_h.at[dstg.at[s]], asv.at[k], semA[k]).wait()

  def compute(s, k):
    def blk(b, __):
      rr = []
      for j in range(3):
        t = tstg[s, j, pl.ds(b * 16, 16)]
        rr.append(jnp.maximum((t - m1[j]) * iv1[j], 0.0))
      for e in range(16):
        row = b * 16 + e
        r0 = rr[0][e]
        r1 = rr[1][e]
        r2 = rr[2][e]
        att = aestg[s, row, :] / (asv[k, row, :] + 1e-16)
        for p in range(8):
          dp = r0 * w2[0][p] + r1 * w2[1][p] + r2 * w2[2][p] + w2[3][p]
          msgb[row, pl.ds(p * 16, 16)] = att * (
              bufH[k, row, pl.ds(p * 16, 16)] + dp)
      return 0

    lax.fori_loop(0, CHE // 16, blk, 0)
    pltpu.sync_copy(msgb, out_sh.at[dstg.at[s]], add=True)

  stage(0, 0)
  stage(1, 1)
  waitstage(0, 0)
  gather(0, 0)

  def six(ci, _):
    g0 = 6 * ci
    for dg in range(6):
      g = g0 + dg
      waitstage(g + 1, (dg + 1) % 3)
      gather((dg + 1) % 3, (dg + 1) % 2)
      stage(g + 2, (dg + 2) % 3)
      waitgather(dg % 3, dg % 2)
      compute(dg % 3, dg % 2)
    return 0

  lax.fori_loop(0, NCHE // 6 - 1, six, 0)
  for g in range(NCHE - 6, NCHE):
    if g + 1 < NCHE:
      waitstage(g + 1, (g + 1) % 3)
      gather((g + 1) % 3, (g + 1) % 2)
    if g + 2 < NCHE:
      stage(g + 2, (g + 2) % 3)
    waitgather(g % 3, g % 2)
    compute(g % 3, g % 2)
  plsc.subcore_barrier()
  for r in range(RPT // 80):
    pltpu.sync_copy(out_sh.at[pl.ds(sid * RPT + r * 80, 80)],
                    msgb.at[pl.ds(0, 80)])
    pltpu.sync_copy(msgb.at[pl.ds(0, 80)],
                    out_o.at[cid, pl.ds(sid * RPT + r * 80, 80)])


def kernel(x, pos, edge_index, W_in, W_out, W_lin, b_lin, W_src, b_src,
           W_dst, b_dst, Wp1, bp1, Wp2, bp2, Wa1, ba1, Wa2, ba2):
  f32 = jnp.float32

  src0, dst0 = edge_index[0], edge_index[1]
  keep = src0 != dst0
  loops = jnp.arange(N, dtype=jnp.int32)
  npad = EP - (E0 + N)
  dump0 = N + (jnp.arange(E0, dtype=jnp.int32) % NDUMP)
  dumpP = N + (jnp.arange(npad, dtype=jnp.int32) % NDUMP)
  src = jnp.concatenate([src0, loops, jnp.zeros((npad,), jnp.int32)])
  dst = jnp.concatenate([jnp.where(keep, dst0, dump0), loops, dumpP])
  msk = jnp.concatenate([keep.astype(f32), jnp.ones((N,), f32),
                         jnp.zeros((npad,), f32)])

  sds = jax.ShapeDtypeStruct
  a_src, a_dst, h_lin = pl.pallas_call(
      _tc_pre,
      out_shape=[sds((N, D), f32)] * 3,
  )(x, W_in.T, W_src.T, b_src[None, :], W_dst.T, b_dst[None, :],
    W_lin.T, b_lin[None, :])

  pad_n = lambda a: jnp.pad(a, ((0, NP - N), (0, 0)))
  a_src_p = pad_n(a_src)
  a_dst_p = pad_n(a_dst)
  hlin_p = pad_n(h_lin)
  posx = jnp.pad(pos[:, 0], (0, NP - N))
  posy = jnp.pad(pos[:, 1], (0, NP - N))
  posz = jnp.pad(pos[:, 2], (0, NP - N))

  prmA = jnp.concatenate([Wp1.reshape(-1), bp1, jnp.zeros((20,), f32)])
  partA, t0a, t1a, t2a = pl.kernel(
      _sc_a,
      out_type=[sds((NW, 112), f32), sds((EP,), f32), sds((EP,), f32),
                sds((EP,), f32)],
      mesh=_mesh,
      compiler_params=pltpu.CompilerParams(needs_layout_passes=False, use_tc_tiling_on_sc=False),
      scratch_types=[
          pltpu.VMEM((NP,), f32), pltpu.VMEM((NP,), f32),
          pltpu.VMEM((NP,), f32),
          pltpu.VMEM((CH,), jnp.int32), pltpu.VMEM((CH,), jnp.int32),
          pltpu.VMEM((CH,), f32),
          pltpu.VMEM((32,), f32), pltpu.VMEM((112,), f32),
          pltpu.VMEM((3, CH), f32),
      ],
  )(posx, posy, posz, src, dst, msk, prmA)

  tot = jnp.sum(partA.reshape(NW, 7, 16), axis=(0, 2))
  n_live = tot[6]
  m1 = tot[0:3] / n_live
  v1 = tot[3:6] / n_live - m1 * m1
  iv1 = lax.rsqrt(v1 + EPS)

  prmB = jnp.concatenate([Wp1.reshape(-1), bp1, jnp.zeros((4,), f32),
                          m1, iv1, jnp.zeros((10,), f32)])
  wp2t = jnp.concatenate([Wp2.T, bp2[None, :]])
  alpha = pl.kernel(
      _sc_b,
      out_type=sds((EP, D), f32),
      mesh=_mesh,
      compiler_params=pltpu.CompilerParams(needs_layout_passes=False, use_tc_tiling_on_sc=False),
      scratch_types=[
          pltpu.VMEM((PTE,), jnp.int32), pltpu.VMEM((PTE,), jnp.int32),
          pltpu.VMEM((PTE,), f32), pltpu.VMEM((3, PTE), f32),
          pltpu.VMEM((32,), f32), pltpu.VMEM((4, D), f32),
          pltpu.VMEM((2, CHB, D), f32), pltpu.VMEM((2, CHB, D), f32),
          pltpu.VMEM((CHB, D), f32),
          pltpu.SemaphoreType.DMA, pltpu.SemaphoreType.DMA,
          pltpu.SemaphoreType.DMA, pltpu.SemaphoreType.DMA,
      ],
  )(a_src_p, a_dst_p, src, dst, msk, t0a, t1a, t2a, prmB, wp2t)

  stats2 = pl.pallas_call(
      _tc_c0,
      grid=(EGRID,),
      in_specs=[pl.BlockSpec((EBLK, D), lambda i: (i, 0))],
      out_specs=pl.BlockSpec((8, D), lambda i: (0, 0)),
      out_shape=sds((8, D), f32),
      scratch_shapes=[pltpu.VMEM((8, D), f32)],
  )(alpha)

  m2 = stats2[0] / n_live
  v2 = stats2[1] / n_live - m2 * m2
  iv2 = lax.rsqrt(v2 + EPS)
  st2 = jnp.zeros((8, D), f32).at[0].set(m2).at[1].set(iv2)

  DA = D // 8
  a1, stats3r = pl.pallas_call(
      _tc_c1,
      grid=(EGRID,),
      in_specs=[
          pl.BlockSpec((EBLK, D), lambda i: (i, 0)),
          pl.BlockSpec((8, D), lambda i: (0, 0)),
          pl.BlockSpec((D, DA), lambda i: (0, 0)),
          pl.BlockSpec((1, DA), lambda i: (0, 0)),
      ],
      out_specs=[
          pl.BlockSpec((EBLK, DA), lambda i: (i, 0)),
          pl.BlockSpec((8, DA), lambda i: (0, 0)),
      ],
      out_shape=[sds((EP, DA), f32), sds((8, DA), f32)],
      scratch_shapes=[pltpu.VMEM((8, DA), f32)],
  )(alpha, st2, Wa1.T, ba1[None, :])

  cdead = jnp.maximum((0.0 - m2) * iv2, 0.0) @ Wa1.T + ba1
  n_dead = jnp.float32(EP) - n_live
  s3 = stats3r[0] - n_dead * cdead
  q3 = stats3r[1] - n_dead * cdead * cdead
  m3 = s3 / n_live
  v3 = q3 / n_live - m3 * m3
  iv3 = lax.rsqrt(v3 + EPS)
  st3 = jnp.zeros((8, DA), f32).at[0].set(m3).at[1].set(iv3)

  ae = pl.pallas_call(
      _tc_c2,
      grid=(EGRID,),
      in_specs=[
          pl.BlockSpec((EBLK, DA), lambda i: (i, 0)),
          pl.BlockSpec((8, DA), lambda i: (0, 0)),
          pl.BlockSpec((DA, DA), lambda i: (0, 0)),
          pl.BlockSpec((1, DA), lambda i: (0, 0)),
      ],
      out_specs=pl.BlockSpec((EBLK, DA), lambda i: (i, 0)),
      out_shape=sds((EP, DA), f32),
  )(a1, st3, Wa2.T, ba2[None, :])

  asum_p, cnt_p = pl.kernel(
      _sc_d,
      out_type=[sds((NC, NP, DA), f32), sds((NC, NP), f32)],
      mesh=_mesh,
      compiler_params=pltpu.CompilerParams(needs_layout_passes=False, use_tc_tiling_on_sc=False),
      scratch_types=[
          pltpu.VMEM_SHARED((NP, DA), f32), pltpu.VMEM_SHARED((NP,), f32),
          pltpu.VMEM((CH, DA), f32),
          pltpu.VMEM((CH,), jnp.int32), pltpu.VMEM((CH,), f32),
          pltpu.VMEM((RPT, DA), f32), pltpu.VMEM((RPT,), f32),
      ],
  )(ae, dst, msk)

  asum = asum_p[0] + asum_p[1]
  cnt = cnt_p[0] + cnt_p[1]

  out_p = pl.kernel(
      _sc_e,
      out_type=sds((NC, NP, D), f32),
      mesh=_mesh,
      compiler_params=pltpu.CompilerParams(needs_layout_passes=False, use_tc_tiling_on_sc=False),
      scratch_types=[
          pltpu.VMEM_SHARED((NP, D), f32),
          pltpu.VMEM((3, CHE), jnp.int32), pltpu.VMEM((3, CHE), jnp.int32),
          pltpu.VMEM((3, 3, CHE), f32), pltpu.VMEM((3, CHE, DA), f32),
          pltpu.VMEM((32,), f32), pltpu.VMEM((4, D), f32),
          pltpu.VMEM((2, CHE, DA), f32),
          pltpu.VMEM((2, CHE, D), f32), pltpu.VMEM((CHE, D), f32),
          pltpu.SemaphoreType.DMA, pltpu.SemaphoreType.DMA,
          pltpu.SemaphoreType.DMA, pltpu.SemaphoreType.DMA,
          pltpu.SemaphoreType.DMA, pltpu.SemaphoreType.DMA,
          pltpu.SemaphoreType.DMA,
      ],
  )(hlin_p, ae, asum, t0a, t1a, t2a, src, dst, prmB, wp2t)

  out = pl.pallas_call(
      _tc_fin,
      out_shape=sds((N, D), f32),
  )(out_p[0, :N], out_p[1, :N], cnt[:N, None], x, W_out.T)
  return out


# speedup vs baseline: 1.4486x; 1.0972x over previous
import functools

import jax
import jax.numpy as jnp
from jax import lax
from jax.experimental import pallas as pl
from jax.experimental.pallas import tpu as pltpu
from jax.experimental.pallas import tpu_sc as plsc

N = 10000
D = 128
E0 = 320000
NP = 10240
NC = 2
NS = 16
NW = NC * NS
PTE = 10368
EP = PTE * NW
CH = 128
NCHUNK = PTE // CH
CHB = 96
NCHB = PTE // CHB
CHE = 96
NCHE = PTE // CHE
RPT = NP // NS
EBLK = 4096
EGRID = EP // EBLK
NDUMP = NP - N
EPS = 1e-5

_mesh = plsc.VectorSubcoreMesh(core_axis_name="c", subcore_axis_name="s")


def _tc_pre(x_ref, wi, ws, bs, wd, bd, wl, bl, as_o, ad_o, hl_o):
  t = jnp.dot(x_ref[...], wi[...], preferred_element_type=jnp.float32)
  m = jnp.mean(t, axis=0, keepdims=True)
  c = t - m
  v = jnp.mean(c * c, axis=0, keepdims=True)
  h = jnp.maximum(c * lax.rsqrt(v + EPS), 0.0)
  as_o[...] = jnp.dot(h, ws[...], preferred_element_type=jnp.float32) + bs[...]
  ad_o[...] = jnp.dot(h, wd[...], preferred_element_type=jnp.float32) + bd[...]
  hl_o[...] = jnp.dot(h, wl[...], preferred_element_type=jnp.float32) + bl[...]


def _tc_c0(al_ref, mk_ref, out_ref, acc):
  i = pl.program_id(0)

  @pl.when(i == 0)
  def _():
    acc[...] = jnp.zeros_like(acc)

  a = al_ref[...] * mk_ref[...]
  acc[0:1, :] += jnp.sum(a, axis=0, keepdims=True)
  acc[1:2, :] += jnp.sum(a * a, axis=0, keepdims=True)

  @pl.when(i == EGRID - 1)
  def _():
    out_ref[...] = acc[...]


def _tc_c1(al_ref, mk_ref, st, wa1, ba1, a1_o, st_o, acc):
  i = pl.program_id(0)

  @pl.when(i == 0)
  def _():
    acc[...] = jnp.zeros_like(acc)

  a = al_ref[...]
  z = jnp.maximum((a - st[0:1, :]) * st[1:2, :], 0.0)
  a1 = jnp.dot(z, wa1[...], preferred_element_type=jnp.float32) + ba1[...]
  a1_o[...] = a1
  a1m = a1 * mk_ref[...]
  acc[0:1, :] += jnp.sum(a1m, axis=0, keepdims=True)
  acc[1:2, :] += jnp.sum(a1m * a1, axis=0, keepdims=True)

  @pl.when(i == EGRID - 1)
  def _():
    st_o[...] = acc[...]


def _tc_c2(a1_ref, st, wa2, ba2, ae_o):
  a1 = a1_ref[...]
  z = jnp.maximum((a1 - st[0:1, :]) * st[1:2, :], 0.0)
  a2 = jnp.dot(z, wa2[...], preferred_element_type=jnp.float32) + ba2[...]
  ae_o[...] = jnp.exp(a2)


def _tc_fin(p0, p1, cnt, x_ref, wo, out_ref):
  s = p0[...] + p1[...]
  o = s / jnp.maximum(cnt[...], 1.0)
  m = jnp.mean(o, axis=0, keepdims=True)
  c = o - m
  v = jnp.mean(c * c, axis=0, keepdims=True)
  o = jnp.maximum(c * lax.rsqrt(v + EPS), 0.0)
  o = jnp.dot(o, wo[...], preferred_element_type=jnp.float32)
  m = jnp.mean(o, axis=0, keepdims=True)
  c = o - m
  v = jnp.mean(c * c, axis=0, keepdims=True)
  o = c * lax.rsqrt(v + EPS)
  out_ref[...] = jnp.maximum(o + x_ref[...], 0.0)


def _wid():
  return lax.axis_index("s") * NC + lax.axis_index("c")


def _pos_rel(px, py, pz, sv, dv):
  rx = plsc.load_gather(px, [dv]) - plsc.load_gather(px, [sv])
  ry = plsc.load_gather(py, [dv]) - plsc.load_gather(py, [sv])
  rz = plsc.load_gather(pz, [dv]) - plsc.load_gather(pz, [sv])
  return rx, ry, rz


def _sc_a(px_h, py_h, pz_h, src_h, dst_h, msk_h, prm_h,
          out_h, t0_h, t1_h, t2_h,
          px, py, pz, sidx, didx, mskv, prm, obuf, tbuf):
  base = _wid() * PTE
  pltpu.sync_copy(px_h, px)
  pltpu.sync_copy(py_h, py)
  pltpu.sync_copy(pz_h, pz)
  pltpu.sync_copy(prm_h, prm)
  va = prm[pl.ds(0, 16)]
  w = [va[i] for i in range(9)]
  bp = [va[9 + i] for i in range(3)]
  zero = jnp.zeros((16,), jnp.float32)

  def chunk(ci, carry):
    off = base + ci * CH
    pltpu.sync_copy(src_h.at[pl.ds(off, CH)], sidx)
    pltpu.sync_copy(dst_h.at[pl.ds(off, CH)], didx)
    pltpu.sync_copy(msk_h.at[pl.ds(off, CH)], mskv)

    def blk(b, c2):
      s0, s1, s2, q0, q1, q2, nl = c2
      sv = sidx[pl.ds(b * 16, 16)]
      dv = didx[pl.ds(b * 16, 16)]
      mv = mskv[pl.ds(b * 16, 16)]
      rx, ry, rz = _pos_rel(px, py, pz, sv, dv)
      t0 = rx * w[0] + ry * w[1] + rz * w[2] + bp[0]
      t1 = rx * w[3] + ry * w[4] + rz * w[5] + bp[1]
      t2 = rx * w[6] + ry * w[7] + rz * w[8] + bp[2]
      tbuf[0, pl.ds(b * 16, 16)] = t0
      tbuf[1, pl.ds(b * 16, 16)] = t1
      tbuf[2, pl.ds(b * 16, 16)] = t2
      tm0 = t0 * mv
      tm1 = t1 * mv
      tm2 = t2 * mv
      return (s0 + tm0, s1 + tm1, s2 + tm2,
              q0 + tm0 * t0, q1 + tm1 * t1, q2 + tm2 * t2, nl + mv)

    carry = lax.fori_loop(0, CH // 16, blk, carry)
    pltpu.sync_copy(tbuf.at[0], t0_h.at[pl.ds(off, CH)])
    pltpu.sync_copy(tbuf.at[1], t1_h.at[pl.ds(off, CH)])
    pltpu.sync_copy(tbuf.at[2], t2_h.at[pl.ds(off, CH)])
    return carry

  acc = lax.fori_loop(0, NCHUNK, chunk, (zero,) * 7)
  for i in range(7):
    obuf[pl.ds(i * 16, 16)] = acc[i]
  pltpu.sync_copy(obuf, out_h.at[_wid()])


def _sc_b(asrc_h, adst_h, src_h, dst_h, t0_h, t1_h, t2_h, prm_h, wp2_h,
          alpha_h,
          sall, dall, tall, prm, wp2, bufS, bufD, abuf,
          semS0, semS1, semD0, semD1, semW0, semW1):
  base = _wid() * PTE
  pltpu.sync_copy(prm_h, prm)
  pltpu.sync_copy(wp2_h, wp2)
  pltpu.sync_copy(src_h.at[pl.ds(base, PTE)], sall)
  pltpu.sync_copy(dst_h.at[pl.ds(base, PTE)], dall)
  vb = prm[pl.ds(16, 16)]
  m1 = [vb[i] for i in range(3)]
  iv1 = [vb[3 + i] for i in range(3)]
  w2 = [[wp2[j, pl.ds(p * 16, 16)] for p in range(8)] for j in range(4)]
  th = [t0_h, t1_h, t2_h]
  semS = [semS0, semS1]
  semD = [semD0, semD1]
  semW = [semW0, semW1]

  def issue(g, k):
    pltpu.async_copy(asrc_h.at[sall.at[pl.ds(g * CHB, CHB)]],
                     bufS.at[k], semS[k])
    pltpu.async_copy(adst_h.at[dall.at[pl.ds(g * CHB, CHB)]],
                     bufD.at[k], semD[k])
    for j in range(3):
      pltpu.async_copy(th[j].at[pl.ds(base + g * CHB, CHB)],
                       tall.at[k, j], semD[k])

  def waitg(g, k):
    pltpu.make_async_copy(asrc_h.at[sall.at[pl.ds(g * CHB, CHB)]],
                          bufS.at[k], semS[k]).wait()
    pltpu.make_async_copy(adst_h.at[dall.at[pl.ds(g * CHB, CHB)]],
                          bufD.at[k], semD[k]).wait()
    for j in range(3):
      pltpu.make_async_copy(th[j].at[pl.ds(base + g * CHB, CHB)],
                            tall.at[k, j], semD[k]).wait()

  def drain_wb(g, k):
    pltpu.make_async_copy(abuf.at[k],
                          alpha_h.at[pl.ds(base + g * CHB, CHB)],
                          semW[k]).wait()

  def compute(g, k):
    def blk(b, __):
      rr = []
      for j in range(3):
        t = tall[k, j, pl.ds(b * 16, 16)]
        rr.append(jnp.maximum((t - m1[j]) * iv1[j], 0.0))
      for e in range(16):
        row = b * 16 + e
        r0 = rr[0][e]
        r1 = rr[1][e]
        r2 = rr[2][e]
        for p in range(8):
          dp = r0 * w2[0][p] + r1 * w2[1][p] + r2 * w2[2][p] + w2[3][p]
          al = (bufD[k, row, pl.ds(p * 16, 16)]
                - bufS[k, row, pl.ds(p * 16, 16)] + dp)
          abuf[k, row, pl.ds(p * 16, 16)] = al
      return 0

    lax.fori_loop(0, CHB // 16, blk, 0)
    pltpu.async_copy(abuf.at[k], alpha_h.at[pl.ds(base + g * CHB, CHB)],
                     semW[k])

  issue(0, 0)
  issue(1, 1)
  waitg(0, 0)
  compute(0, 0)
  issue(2, 0)
  waitg(1, 1)
  compute(1, 1)

  def pair(ci, _):
    g0 = 2 * ci
    issue(g0 + 1, 1)
    waitg(g0, 0)
    drain_wb(g0 - 2, 0)
    compute(g0, 0)
    issue(g0 + 2, 0)
    waitg(g0 + 1, 1)
    drain_wb(g0 - 1, 1)
    compute(g0 + 1, 1)
    return 0

  lax.fori_loop(1, NCHB // 2 - 1, pair, 0)
  issue(NCHB - 1, 1)
  waitg(NCHB - 2, 0)
  drain_wb(NCHB - 4, 0)
  compute(NCHB - 2, 0)
  waitg(NCHB - 1, 1)
  drain_wb(NCHB - 3, 1)
  compute(NCHB - 1, 1)
  drain_wb(NCHB - 2, 0)
  drain_wb(NCHB - 1, 1)


def _sc_d(ae_h, dst_h, msk_h, asum_o, cnt_o,
          asum_sh, cnt_sh, aev, didx, mskv, zb, zb1,
          semT0, semT1, semT2, semC0, semC1, semC2):
  cid = lax.axis_index("c")
  sid = lax.axis_index("s")
  base = _wid() * PTE
  semT = [semT0, semT1, semT2]
  semC = [semC0, semC1, semC2]
  zv = jnp.zeros((16,), jnp.float32)

  def zrow(i, _):
    zb[i, :] = zv
    return 0

  lax.fori_loop(0, RPT, zrow, 0)

  def zrow1(i, _):
    zb1[pl.ds(i * 16, 16)] = zv
    return 0

  lax.fori_loop(0, RPT // 16, zrow1, 0)
  pltpu.sync_copy(zb, asum_sh.at[pl.ds(sid * RPT, RPT)])
  pltpu.sync_copy(zb1, cnt_sh.at[pl.ds(sid * RPT, RPT)])
  plsc.subcore_barrier()

  def stg_copies(g, s):
    off = base + g * CH
    return [
        (ae_h.at[pl.ds(off, CH)], aev.at[s]),
        (dst_h.at[pl.ds(off, CH)], didx.at[s]),
        (msk_h.at[pl.ds(off, CH)], mskv.at[s]),
    ]

  def stage(g, s):
    for a, b in stg_copies(g, s):
      pltpu.async_copy(a, b, semT[s])

  def waitstage(g, s):
    for a, b in stg_copies(g, s):
      pltpu.make_async_copy(a, b, semT[s]).wait()

  def scat(s):
    pltpu.async_copy(aev.at[s], asum_sh.at[didx.at[s]], semC[s], add=True)
    pltpu.async_copy(mskv.at[s], cnt_sh.at[didx.at[s]], semC[s], add=True)

  def waitscat(s):
    pltpu.make_async_copy(aev.at[s], asum_sh.at[didx.at[s]], semC[s]).wait()
    pltpu.make_async_copy(mskv.at[s], cnt_sh.at[didx.at[s]], semC[s]).wait()

  stage(0, 0)
  stage(1, 1)
  waitstage(0, 0)
  scat(0)
  stage(2, 2)
  waitstage(1, 1)
  scat(1)
  waitscat(0)
  stage(3, 0)

  def six(ci, _):
    g0 = 2 + 6 * ci
    for dg in range(6):
      g = g0 + dg
      s = (2 + dg) % 3
      waitstage(g, s)
      scat(s)
      waitscat((1 + dg) % 3)
      stage(g + 2, (1 + dg) % 3)
    return 0

  lax.fori_loop(0, 12, six, 0)
  for g in range(74, NCHUNK):
    s = g % 3
    waitstage(g, s)
    scat(s)
    waitscat((g - 1) % 3)
    if g + 2 < NCHUNK:
      stage(g + 2, (g + 2) % 3)
  waitscat((NCHUNK - 1) % 3)
  plsc.subcore_barrier()
  pltpu.sync_copy(asum_sh.at[pl.ds(sid * RPT, RPT)], zb)
  pltpu.sync_copy(zb, asum_o.at[cid, pl.ds(sid * RPT, RPT)])
  pltpu.sync_copy(cnt_sh.at[pl.ds(sid * RPT, RPT)], zb1)
  pltpu.sync_copy(zb1, cnt_o.at[cid, pl.ds(sid * RPT, RPT)])


def _sc_e(hlin_h, ae_h, asum_h, t0_h, t1_h, t2_h, src_h, dst_h, prm_h, wp2_h,
          out_o,
          out_sh, sstg, dstg, tstg, aestg, prm, wp2, asv, bufH, msgb,
          semH0, semH1, semA0, semA1, semT0, semT1, semT2):
  cid = lax.axis_index("c")
  sid = lax.axis_index("s")
  base = _wid() * PTE
  pltpu.sync_copy(prm_h, prm)
  pltpu.sync_copy(wp2_h, wp2)
  vb = prm[pl.ds(16, 16)]
  m1 = [vb[i] for i in range(3)]
  iv1 = [vb[3 + i] for i in range(3)]
  w2 = [[wp2[j, pl.ds(p * 16, 16)] for p in range(8)] for j in range(4)]
  th = [t0_h, t1_h, t2_h]
  semH = [semH0, semH1]
  semA = [semA0, semA1]
  semT = [semT0, semT1, semT2]
  zv = jnp.zeros((16,), jnp.float32)

  def zrow(i, _):
    for p in range(8):
      msgb[i, pl.ds(p * 16, 16)] = zv
    return 0

  lax.fori_loop(0, CHE, zrow, 0)
  for r in range(RPT // 80):
    pltpu.sync_copy(msgb.at[pl.ds(0, 80)],
                    out_sh.at[pl.ds(sid * RPT + r * 80, 80)])
  plsc.subcore_barrier()

  def stg_copies(g, s):
    off = base + g * CHE
    return [
        (src_h.at[pl.ds(off, CHE)], sstg.at[s]),
        (dst_h.at[pl.ds(off, CHE)], dstg.at[s]),
        (ae_h.at[pl.ds(off, CHE)], aestg.at[s]),
        (th[0].at[pl.ds(off, CHE)], tstg.at[s, 0]),
        (th[1].at[pl.ds(off, CHE)], tstg.at[s, 1]),
        (th[2].at[pl.ds(off, CHE)], tstg.at[s, 2]),
    ]

  def stage(g, s):
    for a, b in stg_copies(g, s):
      pltpu.async_copy(a, b, semT[s])

  def waitstage(g, s):
    for a, b in stg_copies(g, s):
      pltpu.make_async_copy(a, b, semT[s]).wait()

  def gather(s, k):
    pltpu.async_copy(hlin_h.at[sstg.at[s]], bufH.at[k], semH[k])
    pltpu.async_copy(asum_h.at[dstg.at[s]], asv.at[k], semA[k])

  def waitgather(s, k):
    pltpu.make_async_copy(hlin_h.at[sstg.at[s]], bufH.at[k], semH[k]).wait()
    pltpu.make_async_copy(asum_h.at[dstg.at[s]], asv.at[k], semA[k]).wait()

  def compute(s, k):
    def blk(b, __):
      rr = []
      for j in range(3):
        t = tstg[s, j, pl.ds(b * 16, 16)]
        rr.append(jnp.maximum((t - m1[j]) * iv1[j], 0.0))
      for e in range(16):
        row = b * 16 + e
        r0 = rr[0][e]
        r1 = rr[1][e]
        r2 = rr[2][e]
        att = aestg[s, row, :] / (asv[k, row, :] + 1e-16)
        for p in range(8):
          dp = r0 * w2[0][p] + r1 * w2[1][p] + r2 * w2[2][p] + w2[3][p]
          msgb[row, pl.ds(p * 16, 16)] = att * (
              bufH[k, row, pl.ds(p * 16, 16)] + dp)
      return 0

    lax.fori_loop(0, CHE // 16, blk, 0)
    pltpu.sync_copy(msgb, out_sh.at[dstg.at[s]], add=True)

  stage(0, 0)
  stage(1, 1)
  waitstage(0, 0)
  gather(0, 0)

  def six(ci, _):
    g0 = 6 * ci
    for dg in range(6):
      g = g0 + dg
      waitstage(g + 1, (dg + 1) % 3)
      gather((dg + 1) % 3, (dg + 1) % 2)
      stage(g + 2, (dg + 2) % 3)
      waitgather(dg % 3, dg % 2)
      compute(dg % 3, dg % 2)
    return 0

  lax.fori_loop(0, NCHE // 6 - 1, six, 0)
  for g in range(NCHE - 6, NCHE):
    if g + 1 < NCHE:
      waitstage(g + 1, (g + 1) % 3)
      gather((g + 1) % 3, (g + 1) % 2)
    if g + 2 < NCHE:
      stage(g + 2, (g + 2) % 3)
    waitgather(g % 3, g % 2)
    compute(g % 3, g % 2)
  plsc.subcore_barrier()
  for r in range(RPT // 80):
    pltpu.sync_copy(out_sh.at[pl.ds(sid * RPT + r * 80, 80)],
                    msgb.at[pl.ds(0, 80)])
    pltpu.sync_copy(msgb.at[pl.ds(0, 80)],
                    out_o.at[cid, pl.ds(sid * RPT + r * 80, 80)])


def kernel(x, pos, edge_index, W_in, W_out, W_lin, b_lin, W_src, b_src,
           W_dst, b_dst, Wp1, bp1, Wp2, bp2, Wa1, ba1, Wa2, ba2):
  f32 = jnp.float32

  src0, dst0 = edge_index[0], edge_index[1]
  keep = src0 != dst0
  loops = jnp.arange(N, dtype=jnp.int32)
  npad = EP - (E0 + N)
  dump0 = N + (jnp.arange(E0, dtype=jnp.int32) % NDUMP)
  dumpP = N + (jnp.arange(npad, dtype=jnp.int32) % NDUMP)
  src = jnp.concatenate([src0, loops, jnp.zeros((npad,), jnp.int32)])
  dst = jnp.concatenate([jnp.where(keep, dst0, dump0), loops, dumpP])
  msk = jnp.concatenate([keep.astype(f32), jnp.ones((N,), f32),
                         jnp.zeros((npad,), f32)])

  sds = jax.ShapeDtypeStruct
  a_src, a_dst, h_lin = pl.pallas_call(
      _tc_pre,
      out_shape=[sds((N, D), f32)] * 3,
  )(x, W_in.T, W_src.T, b_src[None, :], W_dst.T, b_dst[None, :],
    W_lin.T, b_lin[None, :])

  pad_n = lambda a: jnp.pad(a, ((0, NP - N), (0, 0)))
  a_src_p = pad_n(a_src)
  a_dst_p = pad_n(a_dst)
  hlin_p = pad_n(h_lin)
  posx = jnp.pad(pos[:, 0], (0, NP - N))
  posy = jnp.pad(pos[:, 1], (0, NP - N))
  posz = jnp.pad(pos[:, 2], (0, NP - N))

  prmA = jnp.concatenate([Wp1.reshape(-1), bp1, jnp.zeros((20,), f32)])
  partA, t0a, t1a, t2a = pl.kernel(
      _sc_a,
      out_type=[sds((NW, 112), f32), sds((EP,), f32), sds((EP,), f32),
                sds((EP,), f32)],
      mesh=_mesh,
      compiler_params=pltpu.CompilerParams(needs_layout_passes=False, use_tc_tiling_on_sc=False),
      scratch_types=[
          pltpu.VMEM((NP,), f32), pltpu.VMEM((NP,), f32),
          pltpu.VMEM((NP,), f32),
          pltpu.VMEM((CH,), jnp.int32), pltpu.VMEM((CH,), jnp.int32),
          pltpu.VMEM((CH,), f32),
          pltpu.VMEM((32,), f32), pltpu.VMEM((112,), f32),
          pltpu.VMEM((3, CH), f32),
      ],
  )(posx, posy, posz, src, dst, msk, prmA)

  tot = jnp.sum(partA.reshape(NW, 7, 16), axis=(0, 2))
  n_live = tot[6]
  m1 = tot[0:3] / n_live
  v1 = tot[3:6] / n_live - m1 * m1
  iv1 = lax.rsqrt(v1 + EPS)

  prmB = jnp.concatenate([Wp1.reshape(-1), bp1, jnp.zeros((4,), f32),
                          m1, iv1, jnp.zeros((10,), f32)])
  wp2t = jnp.concatenate([Wp2.T, bp2[None, :]])
  alpha = pl.kernel(
      _sc_b,
      out_type=sds((EP, D), f32),
      mesh=_mesh,
      compiler_params=pltpu.CompilerParams(needs_layout_passes=False, use_tc_tiling_on_sc=False),
      scratch_types=[
          pltpu.VMEM((PTE,), jnp.int32), pltpu.VMEM((PTE,), jnp.int32),
          pltpu.VMEM((2, 3, CHB), f32),
          pltpu.VMEM((32,), f32), pltpu.VMEM((4, D), f32),
          pltpu.VMEM((2, CHB, D), f32), pltpu.VMEM((2, CHB, D), f32),
          pltpu.VMEM((2, CHB, D), f32),
          pltpu.SemaphoreType.DMA, pltpu.SemaphoreType.DMA,
          pltpu.SemaphoreType.DMA, pltpu.SemaphoreType.DMA,
          pltpu.SemaphoreType.DMA, pltpu.SemaphoreType.DMA,
      ],
  )(a_src_p, a_dst_p, src, dst, t0a, t1a, t2a, prmB, wp2t)

  mk2 = msk[:, None]
  stats2 = pl.pallas_call(
      _tc_c0,
      grid=(EGRID,),
      in_specs=[pl.BlockSpec((EBLK, D), lambda i: (i, 0)),
                pl.BlockSpec((EBLK, 1), lambda i: (i, 0))],
      out_specs=pl.BlockSpec((8, D), lambda i: (0, 0)),
      out_shape=sds((8, D), f32),
      scratch_shapes=[pltpu.VMEM((8, D), f32)],
  )(alpha, mk2)

  m2 = stats2[0] / n_live
  v2 = stats2[1] / n_live - m2 * m2
  iv2 = lax.rsqrt(v2 + EPS)
  st2 = jnp.zeros((8, D), f32).at[0].set(m2).at[1].set(iv2)

  DA = D // 8
  a1, stats3r = pl.pallas_call(
      _tc_c1,
      grid=(EGRID,),
      in_specs=[
          pl.BlockSpec((EBLK, D), lambda i: (i, 0)),
          pl.BlockSpec((EBLK, 1), lambda i: (i, 0)),
          pl.BlockSpec((8, D), lambda i: (0, 0)),
          pl.BlockSpec((D, DA), lambda i: (0, 0)),
          pl.BlockSpec((1, DA), lambda i: (0, 0)),
      ],
      out_specs=[
          pl.BlockSpec((EBLK, DA), lambda i: (i, 0)),
          pl.BlockSpec((8, DA), lambda i: (0, 0)),
      ],
      out_shape=[sds((EP, DA), f32), sds((8, DA), f32)],
      scratch_shapes=[pltpu.VMEM((8, DA), f32)],
  )(alpha, mk2, st2, Wa1.T, ba1[None, :])

  m3 = stats3r[0] / n_live
  v3 = stats3r[1] / n_live - m3 * m3
  iv3 = lax.rsqrt(v3 + EPS)
  st3 = jnp.zeros((8, DA), f32).at[0].set(m3).at[1].set(iv3)

  ae = pl.pallas_call(
      _tc_c2,
      grid=(EGRID,),
      in_specs=[
          pl.BlockSpec((EBLK, DA), lambda i: (i, 0)),
          pl.BlockSpec((8, DA), lambda i: (0, 0)),
          pl.BlockSpec((DA, DA), lambda i: (0, 0)),
          pl.BlockSpec((1, DA), lambda i: (0, 0)),
      ],
      out_specs=pl.BlockSpec((EBLK, DA), lambda i: (i, 0)),
      out_shape=sds((EP, DA), f32),
  )(a1, st3, Wa2.T, ba2[None, :])

  asum_p, cnt_p = pl.kernel(
      _sc_d,
      out_type=[sds((NC, NP, DA), f32), sds((NC, NP), f32)],
      mesh=_mesh,
      compiler_params=pltpu.CompilerParams(needs_layout_passes=False, use_tc_tiling_on_sc=False),
      scratch_types=[
          pltpu.VMEM_SHARED((NP, DA), f32), pltpu.VMEM_SHARED((NP,), f32),
          pltpu.VMEM((3, CH, DA), f32),
          pltpu.VMEM((3, CH), jnp.int32), pltpu.VMEM((3, CH), f32),
          pltpu.VMEM((RPT, DA), f32), pltpu.VMEM((RPT,), f32),
          pltpu.SemaphoreType.DMA, pltpu.SemaphoreType.DMA,
          pltpu.SemaphoreType.DMA, pltpu.SemaphoreType.DMA,
          pltpu.SemaphoreType.DMA, pltpu.SemaphoreType.DMA,
      ],
  )(ae, dst, msk)

  asum = asum_p[0] + asum_p[1]
  cnt = cnt_p[0] + cnt_p[1]

  out_p = pl.kernel(
      _sc_e,
      out_type=sds((NC, NP, D), f32),
      mesh=_mesh,
      compiler_params=pltpu.CompilerParams(needs_layout_passes=False, use_tc_tiling_on_sc=False),
      scratch_types=[
          pltpu.VMEM_SHARED((NP, D), f32),
          pltpu.VMEM((3, CHE), jnp.int32), pltpu.VMEM((3, CHE), jnp.int32),
          pltpu.VMEM((3, 3, CHE), f32), pltpu.VMEM((3, CHE, DA), f32),
          pltpu.VMEM((32,), f32), pltpu.VMEM((4, D), f32),
          pltpu.VMEM((2, CHE, DA), f32),
          pltpu.VMEM((2, CHE, D), f32), pltpu.VMEM((CHE, D), f32),
          pltpu.SemaphoreType.DMA, pltpu.SemaphoreType.DMA,
          pltpu.SemaphoreType.DMA, pltpu.SemaphoreType.DMA,
          pltpu.SemaphoreType.DMA, pltpu.SemaphoreType.DMA,
          pltpu.SemaphoreType.DMA,
      ],
  )(hlin_p, ae, asum, t0a, t1a, t2a, src, dst, prmB, wp2t)

  out = pl.pallas_call(
      _tc_fin,
      out_shape=sds((N, D), f32),
  )(out_p[0, :N], out_p[1, :N], cnt[:N, None], x, W_out.T)
  return out


# speedup vs baseline: 1.5694x; 1.0834x over previous
import functools

import jax
import jax.numpy as jnp
from jax import lax
from jax.experimental import pallas as pl
from jax.experimental.pallas import tpu as pltpu
from jax.experimental.pallas import tpu_sc as plsc

N = 10000
D = 128
E0 = 320000
NP = 10240
NC = 2
NS = 16
NW = NC * NS
PTE = 10368
EP = PTE * NW
CH = 128
NCHUNK = PTE // CH
CHB = 96
NCHB = PTE // CHB
CHE = 96
NCHE = PTE // CHE
RPT = NP // NS
EBLK = 4096
EGRID = EP // EBLK
NDUMP = NP - N
EPS = 1e-5

_mesh = plsc.VectorSubcoreMesh(core_axis_name="c", subcore_axis_name="s")


def _tc_pre(x_ref, wi, ws, bs, wd, bd, wl, bl, as_o, ad_o, hl_o):
  t = jnp.dot(x_ref[...], wi[...], preferred_element_type=jnp.float32)
  m = jnp.mean(t, axis=0, keepdims=True)
  c = t - m
  v = jnp.mean(c * c, axis=0, keepdims=True)
  h = jnp.maximum(c * lax.rsqrt(v + EPS), 0.0)
  as_o[...] = jnp.dot(h, ws[...], preferred_element_type=jnp.float32) + bs[...]
  ad_o[...] = jnp.dot(h, wd[...], preferred_element_type=jnp.float32) + bd[...]
  hl_o[...] = jnp.dot(h, wl[...], preferred_element_type=jnp.float32) + bl[...]


def _tc_c0(al_ref, mk_ref, out_ref, acc):
  i = pl.program_id(0)

  @pl.when(i == 0)
  def _():
    acc[...] = jnp.zeros_like(acc)

  a = al_ref[...] * mk_ref[...]
  acc[0:1, :] += jnp.sum(a, axis=0, keepdims=True)
  acc[1:2, :] += jnp.sum(a * a, axis=0, keepdims=True)

  @pl.when(i == EGRID - 1)
  def _():
    out_ref[...] = acc[...]


def _tc_c1(al_ref, mk_ref, st, wa1, ba1, a1_o, st_o, acc):
  i = pl.program_id(0)

  @pl.when(i == 0)
  def _():
    acc[...] = jnp.zeros_like(acc)

  a = al_ref[...]
  z = jnp.maximum((a - st[0:1, :]) * st[1:2, :], 0.0)
  a1 = jnp.dot(z, wa1[...], preferred_element_type=jnp.float32) + ba1[...]
  a1_o[...] = a1
  a1m = a1 * mk_ref[...]
  acc[0:1, :] += jnp.sum(a1m, axis=0, keepdims=True)
  acc[1:2, :] += jnp.sum(a1m * a1, axis=0, keepdims=True)

  @pl.when(i == EGRID - 1)
  def _():
    st_o[...] = acc[...]


def _tc_c2(a1_ref, st, wa2, ba2, ae_o):
  a1 = a1_ref[...]
  z = jnp.maximum((a1 - st[0:1, :]) * st[1:2, :], 0.0)
  a2 = jnp.dot(z, wa2[...], preferred_element_type=jnp.float32) + ba2[...]
  ae_o[...] = jnp.exp(a2)


def _tc_fin(p0, p1, cnt, x_ref, wo, out_ref):
  s = p0[...] + p1[...]
  o = s / jnp.maximum(cnt[...], 1.0)
  m = jnp.mean(o, axis=0, keepdims=True)
  c = o - m
  v = jnp.mean(c * c, axis=0, keepdims=True)
  o = jnp.maximum(c * lax.rsqrt(v + EPS), 0.0)
  o = jnp.dot(o, wo[...], preferred_element_type=jnp.float32)
  m = jnp.mean(o, axis=0, keepdims=True)
  c = o - m
  v = jnp.mean(c * c, axis=0, keepdims=True)
  o = c * lax.rsqrt(v + EPS)
  out_ref[...] = jnp.maximum(o + x_ref[...], 0.0)


def _wid():
  return lax.axis_index("s") * NC + lax.axis_index("c")


def _pos_rel(px, py, pz, sv, dv):
  rx = plsc.load_gather(px, [dv]) - plsc.load_gather(px, [sv])
  ry = plsc.load_gather(py, [dv]) - plsc.load_gather(py, [sv])
  rz = plsc.load_gather(pz, [dv]) - plsc.load_gather(pz, [sv])
  return rx, ry, rz


def _sc_a(px_h, py_h, pz_h, src_h, dst_h, msk_h, prm_h,
          out_h, t0_h, t1_h, t2_h,
          px, py, pz, sall, dall, mall, prm, obuf, tbuf, semW0, semW1):
  base = _wid() * PTE
  pltpu.sync_copy(px_h, px)
  pltpu.sync_copy(py_h, py)
  pltpu.sync_copy(pz_h, pz)
  pltpu.sync_copy(prm_h, prm)
  pltpu.sync_copy(src_h.at[pl.ds(base, PTE)], sall)
  pltpu.sync_copy(dst_h.at[pl.ds(base, PTE)], dall)
  pltpu.sync_copy(msk_h.at[pl.ds(base, PTE)], mall)
  va = prm[pl.ds(0, 16)]
  w = [va[i] for i in range(9)]
  bp = [va[9 + i] for i in range(3)]
  th = [t0_h, t1_h, t2_h]
  semW = [semW0, semW1]
  zero = jnp.zeros((16,), jnp.float32)

  def drain_wb(g, k):
    for j in range(3):
      pltpu.make_async_copy(tbuf.at[k, j],
                            th[j].at[pl.ds(base + g * CH, CH)],
                            semW[k]).wait()

  def compute(g, k, carry):
    def blk(b, c2):
      s0, s1, s2, q0, q1, q2, nl = c2
      lo = g * CH + b * 16
      sv = sall[pl.ds(lo, 16)]
      dv = dall[pl.ds(lo, 16)]
      mv = mall[pl.ds(lo, 16)]
      rx, ry, rz = _pos_rel(px, py, pz, sv, dv)
      t0 = rx * w[0] + ry * w[1] + rz * w[2] + bp[0]
      t1 = rx * w[3] + ry * w[4] + rz * w[5] + bp[1]
      t2 = rx * w[6] + ry * w[7] + rz * w[8] + bp[2]
      tbuf[k, 0, pl.ds(b * 16, 16)] = t0
      tbuf[k, 1, pl.ds(b * 16, 16)] = t1
      tbuf[k, 2, pl.ds(b * 16, 16)] = t2
      tm0 = t0 * mv
      tm1 = t1 * mv
      tm2 = t2 * mv
      return (s0 + tm0, s1 + tm1, s2 + tm2,
              q0 + tm0 * t0, q1 + tm1 * t1, q2 + tm2 * t2, nl + mv)

    carry = lax.fori_loop(0, CH // 16, blk, carry)
    for j in range(3):
      pltpu.async_copy(tbuf.at[k, j], th[j].at[pl.ds(base + g * CH, CH)],
                       semW[k])
    return carry

  acc = compute(0, 0, (zero,) * 7)
  acc = compute(1, 1, acc)

  def pair(ci, carry):
    g0 = 2 * ci
    drain_wb(g0 - 2, 0)
    carry = compute(g0, 0, carry)
    drain_wb(g0 - 1, 1)
    carry = compute(g0 + 1, 1, carry)
    return carry

  acc = lax.fori_loop(1, NCHUNK // 2, pair, acc)
  drain_wb(NCHUNK - 3, 0)
  acc = compute(NCHUNK - 1, 0, acc)
  drain_wb(NCHUNK - 2, 1)
  drain_wb(NCHUNK - 1, 0)
  for i in range(7):
    obuf[pl.ds(i * 16, 16)] = acc[i]
  pltpu.sync_copy(obuf, out_h.at[_wid()])


def _sc_b(asrc_h, adst_h, src_h, dst_h, t0_h, t1_h, t2_h, prm_h, wp2_h,
          alpha_h,
          sall, dall, tall, prm, wp2, bufS, bufD, abuf,
          semS0, semS1, semD0, semD1, semW0, semW1):
  base = _wid() * PTE
  pltpu.sync_copy(prm_h, prm)
  pltpu.sync_copy(wp2_h, wp2)
  pltpu.sync_copy(src_h.at[pl.ds(base, PTE)], sall)
  pltpu.sync_copy(dst_h.at[pl.ds(base, PTE)], dall)
  vb = prm[pl.ds(16, 16)]
  m1 = [vb[i] for i in range(3)]
  iv1 = [vb[3 + i] for i in range(3)]
  w2 = [[wp2[j, pl.ds(p * 16, 16)] for p in range(8)] for j in range(4)]
  th = [t0_h, t1_h, t2_h]
  semS = [semS0, semS1]
  semD = [semD0, semD1]
  semW = [semW0, semW1]

  def issue(g, k):
    pltpu.async_copy(asrc_h.at[sall.at[pl.ds(g * CHB, CHB)]],
                     bufS.at[k], semS[k])
    pltpu.async_copy(adst_h.at[dall.at[pl.ds(g * CHB, CHB)]],
                     bufD.at[k], semD[k])
    for j in range(3):
      pltpu.async_copy(th[j].at[pl.ds(base + g * CHB, CHB)],
                       tall.at[k, j], semD[k])

  def waitg(g, k):
    pltpu.make_async_copy(asrc_h.at[sall.at[pl.ds(g * CHB, CHB)]],
                          bufS.at[k], semS[k]).wait()
    pltpu.make_async_copy(adst_h.at[dall.at[pl.ds(g * CHB, CHB)]],
                          bufD.at[k], semD[k]).wait()
    for j in range(3):
      pltpu.make_async_copy(th[j].at[pl.ds(base + g * CHB, CHB)],
                            tall.at[k, j], semD[k]).wait()

  def drain_wb(g, k):
    pltpu.make_async_copy(abuf.at[k],
                          alpha_h.at[pl.ds(base + g * CHB, CHB)],
                          semW[k]).wait()

  def compute(g, k):
    def blk(b, __):
      rr = []
      for j in range(3):
        t = tall[k, j, pl.ds(b * 16, 16)]
        rr.append(jnp.maximum((t - m1[j]) * iv1[j], 0.0))
      for e in range(16):
        row = b * 16 + e
        r0 = rr[0][e]
        r1 = rr[1][e]
        r2 = rr[2][e]
        for p in range(8):
          dp = r0 * w2[0][p] + r1 * w2[1][p] + r2 * w2[2][p] + w2[3][p]
          al = (bufD[k, row, pl.ds(p * 16, 16)]
                - bufS[k, row, pl.ds(p * 16, 16)] + dp)
          abuf[k, row, pl.ds(p * 16, 16)] = al
      return 0

    lax.fori_loop(0, CHB // 16, blk, 0)
    pltpu.async_copy(abuf.at[k], alpha_h.at[pl.ds(base + g * CHB, CHB)],
                     semW[k])

  issue(0, 0)
  issue(1, 1)
  waitg(0, 0)
  compute(0, 0)
  issue(2, 0)
  waitg(1, 1)
  compute(1, 1)

  def pair(ci, _):
    g0 = 2 * ci
    issue(g0 + 1, 1)
    waitg(g0, 0)
    drain_wb(g0 - 2, 0)
    compute(g0, 0)
    issue(g0 + 2, 0)
    waitg(g0 + 1, 1)
    drain_wb(g0 - 1, 1)
    compute(g0 + 1, 1)
    return 0

  lax.fori_loop(1, NCHB // 2 - 1, pair, 0)
  issue(NCHB - 1, 1)
  waitg(NCHB - 2, 0)
  drain_wb(NCHB - 4, 0)
  compute(NCHB - 2, 0)
  waitg(NCHB - 1, 1)
  drain_wb(NCHB - 3, 1)
  compute(NCHB - 1, 1)
  drain_wb(NCHB - 2, 0)
  drain_wb(NCHB - 1, 1)


def _sc_d(ae_h, dst_h, msk_h, asum_o, cnt_o,
          asum_sh, cnt_sh, aev, didx, mskv, zb, zb1,
          semT0, semT1, semT2, semC0, semC1, semC2):
  cid = lax.axis_index("c")
  sid = lax.axis_index("s")
  base = _wid() * PTE
  semT = [semT0, semT1, semT2]
  semC = [semC0, semC1, semC2]
  zv = jnp.zeros((16,), jnp.float32)

  def zrow(i, _):
    zb[i, :] = zv
    return 0

  lax.fori_loop(0, RPT, zrow, 0)

  def zrow1(i, _):
    zb1[pl.ds(i * 16, 16)] = zv
    return 0

  lax.fori_loop(0, RPT // 16, zrow1, 0)
  pltpu.sync_copy(zb, asum_sh.at[pl.ds(sid * RPT, RPT)])
  pltpu.sync_copy(zb1, cnt_sh.at[pl.ds(sid * RPT, RPT)])
  plsc.subcore_barrier()

  def stg_copies(g, s):
    off = base + g * CH
    return [
        (ae_h.at[pl.ds(off, CH)], aev.at[s]),
        (dst_h.at[pl.ds(off, CH)], didx.at[s]),
        (msk_h.at[pl.ds(off, CH)], mskv.at[s]),
    ]

  def stage(g, s):
    for a, b in stg_copies(g, s):
      pltpu.async_copy(a, b, semT[s])

  def waitstage(g, s):
    for a, b in stg_copies(g, s):
      pltpu.make_async_copy(a, b, semT[s]).wait()

  def scat(s):
    pltpu.async_copy(aev.at[s], asum_sh.at[didx.at[s]], semC[s], add=True)
    pltpu.async_copy(mskv.at[s], cnt_sh.at[didx.at[s]], semC[s], add=True)

  def waitscat(s):
    pltpu.make_async_copy(aev.at[s], asum_sh.at[didx.at[s]], semC[s]).wait()
    pltpu.make_async_copy(mskv.at[s], cnt_sh.at[didx.at[s]], semC[s]).wait()

  stage(0, 0)
  stage(1, 1)
  waitstage(0, 0)
  scat(0)
  stage(2, 2)
  waitstage(1, 1)
  scat(1)
  waitscat(0)
  stage(3, 0)

  def six(ci, _):
    g0 = 2 + 6 * ci
    for dg in range(6):
      g = g0 + dg
      s = (2 + dg) % 3
      waitstage(g, s)
      scat(s)
      waitscat((1 + dg) % 3)
      stage(g + 2, (1 + dg) % 3)
    return 0

  lax.fori_loop(0, 12, six, 0)
  for g in range(74, NCHUNK):
    s = g % 3
    waitstage(g, s)
    scat(s)
    waitscat((g - 1) % 3)
    if g + 2 < NCHUNK:
      stage(g + 2, (g + 2) % 3)
  waitscat((NCHUNK - 1) % 3)
  plsc.subcore_barrier()
  pltpu.sync_copy(asum_sh.at[pl.ds(sid * RPT, RPT)], zb)
  pltpu.sync_copy(zb, asum_o.at[cid, pl.ds(sid * RPT, RPT)])
  pltpu.sync_copy(cnt_sh.at[pl.ds(sid * RPT, RPT)], zb1)
  pltpu.sync_copy(zb1, cnt_o.at[cid, pl.ds(sid * RPT, RPT)])


def _sc_e(hlin_h, ae_h, asum_h, t0_h, t1_h, t2_h, src_h, dst_h, prm_h, wp2_h,
          out_o,
          out_sh, sstg, dstg, tstg, aestg, prm, wp2, asv, bufH, msgb,
          semH0, semH1, semA0, semA1, semT0, semT1, semT2):
  cid = lax.axis_index("c")
  sid = lax.axis_index("s")
  base = _wid() * PTE
  pltpu.sync_copy(prm_h, prm)
  pltpu.sync_copy(wp2_h, wp2)
  vb = prm[pl.ds(16, 16)]
  m1 = [vb[i] for i in range(3)]
  iv1 = [vb[3 + i] for i in range(3)]
  w2 = [[wp2[j, pl.ds(p * 16, 16)] for p in range(8)] for j in range(4)]
  th = [t0_h, t1_h, t2_h]
  semH = [semH0, semH1]
  semA = [semA0, semA1]
  semT = [semT0, semT1, semT2]
  zv = jnp.zeros((16,), jnp.float32)

  def zrow(i, _):
    for p in range(8):
      msgb[i, pl.ds(p * 16, 16)] = zv
    return 0

  lax.fori_loop(0, CHE, zrow, 0)
  for r in range(RPT // 80):
    pltpu.sync_copy(msgb.at[pl.ds(0, 80)],
                    out_sh.at[pl.ds(sid * RPT + r * 80, 80)])
  plsc.subcore_barrier()

  def stg_copies(g, s):
    off = base + g * CHE
    return [
        (src_h.at[pl.ds(off, CHE)], sstg.at[s]),
        (dst_h.at[pl.ds(off, CHE)], dstg.at[s]),
        (ae_h.at[pl.ds(off, CHE)], aestg.at[s]),
        (th[0].at[pl.ds(off, CHE)], tstg.at[s, 0]),
        (th[1].at[pl.ds(off, CHE)], tstg.at[s, 1]),
        (th[2].at[pl.ds(off, CHE)], tstg.at[s, 2]),
    ]

  def stage(g, s):
    for a, b in stg_copies(g, s):
      pltpu.async_copy(a, b, semT[s])

  def waitstage(g, s):
    for a, b in stg_copies(g, s):
      pltpu.make_async_copy(a, b, semT[s]).wait()

  def gather(s, k):
    pltpu.async_copy(hlin_h.at[sstg.at[s]], bufH.at[k], semH[k])
    pltpu.async_copy(asum_h.at[dstg.at[s]], asv.at[k], semA[k])

  def waitgather(s, k):
    pltpu.make_async_copy(hlin_h.at[sstg.at[s]], bufH.at[k], semH[k]).wait()
    pltpu.make_async_copy(asum_h.at[dstg.at[s]], asv.at[k], semA[k]).wait()

  def compute(s, k):
    def blk(b, __):
      rr = []
      for j in range(3):
        t = tstg[s, j, pl.ds(b * 16, 16)]
        rr.append(jnp.maximum((t - m1[j]) * iv1[j], 0.0))
      for e in range(16):
        row = b * 16 + e
        r0 = rr[0][e]
        r1 = rr[1][e]
        r2 = rr[2][e]
        att = aestg[s, row, :] / (asv[k, row, :] + 1e-16)
        for p in range(8):
          dp = r0 * w2[0][p] + r1 * w2[1][p] + r2 * w2[2][p] + w2[3][p]
          msgb[row, pl.ds(p * 16, 16)] = att * (
              bufH[k, row, pl.ds(p * 16, 16)] + dp)
      return 0

    lax.fori_loop(0, CHE // 16, blk, 0)
    pltpu.sync_copy(msgb, out_sh.at[dstg.at[s]], add=True)

  stage(0, 0)
  stage(1, 1)
  waitstage(0, 0)
  gather(0, 0)

  def six(ci, _):
    g0 = 6 * ci
    for dg in range(6):
      g = g0 + dg
      waitstage(g + 1, (dg + 1) % 3)
      gather((dg + 1) % 3, (dg + 1) % 2)
      stage(g + 2, (dg + 2) % 3)
      waitgather(dg % 3, dg % 2)
      compute(dg % 3, dg % 2)
    return 0

  lax.fori_loop(0, NCHE // 6 - 1, six, 0)
  for g in range(NCHE - 6, NCHE):
    if g + 1 < NCHE:
      waitstage(g + 1, (g + 1) % 3)
      gather((g + 1) % 3, (g + 1) % 2)
    if g + 2 < NCHE:
      stage(g + 2, (g + 2) % 3)
    waitgather(g % 3, g % 2)
    compute(g % 3, g % 2)
  plsc.subcore_barrier()
  for r in range(RPT // 80):
    pltpu.sync_copy(out_sh.at[pl.ds(sid * RPT + r * 80, 80)],
                    msgb.at[pl.ds(0, 80)])
    pltpu.sync_copy(msgb.at[pl.ds(0, 80)],
                    out_o.at[cid, pl.ds(sid * RPT + r * 80, 80)])


def kernel(x, pos, edge_index, W_in, W_out, W_lin, b_lin, W_src, b_src,
           W_dst, b_dst, Wp1, bp1, Wp2, bp2, Wa1, ba1, Wa2, ba2):
  f32 = jnp.float32

  src0, dst0 = edge_index[0], edge_index[1]
  keep = src0 != dst0
  loops = jnp.arange(N, dtype=jnp.int32)
  npad = EP - (E0 + N)
  dump0 = N + (jnp.arange(E0, dtype=jnp.int32) % NDUMP)
  dumpP = N + (jnp.arange(npad, dtype=jnp.int32) % NDUMP)
  src = jnp.concatenate([src0, loops, jnp.zeros((npad,), jnp.int32)])
  dst = jnp.concatenate([jnp.where(keep, dst0, dump0), loops, dumpP])
  msk = jnp.concatenate([keep.astype(f32), jnp.ones((N,), f32),
                         jnp.zeros((npad,), f32)])

  sds = jax.ShapeDtypeStruct
  a_src, a_dst, h_lin = pl.pallas_call(
      _tc_pre,
      out_shape=[sds((N, D), f32)] * 3,
  )(x, W_in.T, W_src.T, b_src[None, :], W_dst.T, b_dst[None, :],
    W_lin.T, b_lin[None, :])

  pad_n = lambda a: jnp.pad(a, ((0, NP - N), (0, 0)))
  a_src_p = pad_n(a_src)
  a_dst_p = pad_n(a_dst)
  hlin_p = pad_n(h_lin)
  posx = jnp.pad(pos[:, 0], (0, NP - N))
  posy = jnp.pad(pos[:, 1], (0, NP - N))
  posz = jnp.pad(pos[:, 2], (0, NP - N))

  prmA = jnp.concatenate([Wp1.reshape(-1), bp1, jnp.zeros((20,), f32)])
  partA, t0a, t1a, t2a = pl.kernel(
      _sc_a,
      out_type=[sds((NW, 112), f32), sds((EP,), f32), sds((EP,), f32),
                sds((EP,), f32)],
      mesh=_mesh,
      compiler_params=pltpu.CompilerParams(needs_layout_passes=False, use_tc_tiling_on_sc=False),
      scratch_types=[
          pltpu.VMEM((NP,), f32), pltpu.VMEM((NP,), f32),
          pltpu.VMEM((NP,), f32),
          pltpu.VMEM((PTE,), jnp.int32), pltpu.VMEM((PTE,), jnp.int32),
          pltpu.VMEM((PTE,), f32),
          pltpu.VMEM((32,), f32), pltpu.VMEM((112,), f32),
          pltpu.VMEM((2, 3, CH), f32),
          pltpu.SemaphoreType.DMA, pltpu.SemaphoreType.DMA,
      ],
  )(posx, posy, posz, src, dst, msk, prmA)

  tot = jnp.sum(partA.reshape(NW, 7, 16), axis=(0, 2))
  n_live = tot[6]
  m1 = tot[0:3] / n_live
  v1 = tot[3:6] / n_live - m1 * m1
  iv1 = lax.rsqrt(v1 + EPS)

  prmB = jnp.concatenate([Wp1.reshape(-1), bp1, jnp.zeros((4,), f32),
                          m1, iv1, jnp.zeros((10,), f32)])
  wp2t = jnp.concatenate([Wp2.T, bp2[None, :]])
  alpha = pl.kernel(
      _sc_b,
      out_type=sds((EP, D), f32),
      mesh=_mesh,
      compiler_params=pltpu.CompilerParams(needs_layout_passes=False, use_tc_tiling_on_sc=False),
      scratch_types=[
          pltpu.VMEM((PTE,), jnp.int32), pltpu.VMEM((PTE,), jnp.int32),
          pltpu.VMEM((2, 3, CHB), f32),
          pltpu.VMEM((32,), f32), pltpu.VMEM((4, D), f32),
          pltpu.VMEM((2, CHB, D), f32), pltpu.VMEM((2, CHB, D), f32),
          pltpu.VMEM((2, CHB, D), f32),
          pltpu.SemaphoreType.DMA, pltpu.SemaphoreType.DMA,
          pltpu.SemaphoreType.DMA, pltpu.SemaphoreType.DMA,
          pltpu.SemaphoreType.DMA, pltpu.SemaphoreType.DMA,
      ],
  )(a_src_p, a_dst_p, src, dst, t0a, t1a, t2a, prmB, wp2t)

  mk2 = msk[:, None]
  stats2 = pl.pallas_call(
      _tc_c0,
      grid=(EGRID,),
      in_specs=[pl.BlockSpec((EBLK, D), lambda i: (i, 0)),
                pl.BlockSpec((EBLK, 1), lambda i: (i, 0))],
      out_specs=pl.BlockSpec((8, D), lambda i: (0, 0)),
      out_shape=sds((8, D), f32),
      scratch_shapes=[pltpu.VMEM((8, D), f32)],
  )(alpha, mk2)

  m2 = stats2[0] / n_live
  v2 = stats2[1] / n_live - m2 * m2
  iv2 = lax.rsqrt(v2 + EPS)
  st2 = jnp.zeros((8, D), f32).at[0].set(m2).at[1].set(iv2)

  DA = D // 8
  a1, stats3r = pl.pallas_call(
      _tc_c1,
      grid=(EGRID,),
      in_specs=[
          pl.BlockSpec((EBLK, D), lambda i: (i, 0)),
          pl.BlockSpec((EBLK, 1), lambda i: (i, 0)),
          pl.BlockSpec((8, D), lambda i: (0, 0)),
          pl.BlockSpec((D, DA), lambda i: (0, 0)),
          pl.BlockSpec((1, DA), lambda i: (0, 0)),
      ],
      out_specs=[
          pl.BlockSpec((EBLK, DA), lambda i: (i, 0)),
          pl.BlockSpec((8, DA), lambda i: (0, 0)),
      ],
      out_shape=[sds((EP, DA), f32), sds((8, DA), f32)],
      scratch_shapes=[pltpu.VMEM((8, DA), f32)],
  )(alpha, mk2, st2, Wa1.T, ba1[None, :])

  m3 = stats3r[0] / n_live
  v3 = stats3r[1] / n_live - m3 * m3
  iv3 = lax.rsqrt(v3 + EPS)
  st3 = jnp.zeros((8, DA), f32).at[0].set(m3).at[1].set(iv3)

  ae = pl.pallas_call(
      _tc_c2,
      grid=(EGRID,),
      in_specs=[
          pl.BlockSpec((EBLK, DA), lambda i: (i, 0)),
          pl.BlockSpec((8, DA), lambda i: (0, 0)),
          pl.BlockSpec((DA, DA), lambda i: (0, 0)),
          pl.BlockSpec((1, DA), lambda i: (0, 0)),
      ],
      out_specs=pl.BlockSpec((EBLK, DA), lambda i: (i, 0)),
      out_shape=sds((EP, DA), f32),
  )(a1, st3, Wa2.T, ba2[None, :])

  asum_p, cnt_p = pl.kernel(
      _sc_d,
      out_type=[sds((NC, NP, DA), f32), sds((NC, NP), f32)],
      mesh=_mesh,
      compiler_params=pltpu.CompilerParams(needs_layout_passes=False, use_tc_tiling_on_sc=False),
      scratch_types=[
          pltpu.VMEM_SHARED((NP, DA), f32), pltpu.VMEM_SHARED((NP,), f32),
          pltpu.VMEM((3, CH, DA), f32),
          pltpu.VMEM((3, CH), jnp.int32), pltpu.VMEM((3, CH), f32),
          pltpu.VMEM((RPT, DA), f32), pltpu.VMEM((RPT,), f32),
          pltpu.SemaphoreType.DMA, pltpu.SemaphoreType.DMA,
          pltpu.SemaphoreType.DMA, pltpu.SemaphoreType.DMA,
          pltpu.SemaphoreType.DMA, pltpu.SemaphoreType.DMA,
      ],
  )(ae, dst, msk)

  asum = asum_p[0] + asum_p[1]
  cnt = cnt_p[0] + cnt_p[1]

  out_p = pl.kernel(
      _sc_e,
      out_type=sds((NC, NP, D), f32),
      mesh=_mesh,
      compiler_params=pltpu.CompilerParams(needs_layout_passes=False, use_tc_tiling_on_sc=False),
      scratch_types=[
          pltpu.VMEM_SHARED((NP, D), f32),
          pltpu.VMEM((3, CHE), jnp.int32), pltpu.VMEM((3, CHE), jnp.int32),
          pltpu.VMEM((3, 3, CHE), f32), pltpu.VMEM((3, CHE, DA), f32),
          pltpu.VMEM((32,), f32), pltpu.VMEM((4, D), f32),
          pltpu.VMEM((2, CHE, DA), f32),
          pltpu.VMEM((2, CHE, D), f32), pltpu.VMEM((CHE, D), f32),
          pltpu.SemaphoreType.DMA, pltpu.SemaphoreType.DMA,
          pltpu.SemaphoreType.DMA, pltpu.SemaphoreType.DMA,
          pltpu.SemaphoreType.DMA, pltpu.SemaphoreType.DMA,
          pltpu.SemaphoreType.DMA,
      ],
  )(hlin_p, ae, asum, t0a, t1a, t2a, src, dst, prmB, wp2t)

  out = pl.pallas_call(
      _tc_fin,
      out_shape=sds((N, D), f32),
  )(out_p[0, :N], out_p[1, :N], cnt[:N, None], x, W_out.T)
  return out
